# Initial kernel scaffold; baseline (speedup 1.0000x reference)
import jax
import jax.numpy as jnp
from jax.experimental import pallas as pl


def kernel(x, edge_index, e, params):
    raise NotImplementedError("write your pallas kernel here")


# speedup vs baseline: 10.9090x; 10.9090x over previous
import functools
import math

import numpy as np
import jax
import jax.numpy as jnp
from jax import lax
from jax.experimental import pallas as pl
from jax.experimental.pallas import tpu as pltpu
from jax.experimental.pallas import tpu_sc as plsc

N = 10000
E = 320000
D = 128
DE = 16
H = 4
DN_H = 32
DE_H = 4

NC = 2
NS = 16
NW = NC * NS
CH = 128
GRAN = NW * CH

_f32 = jnp.float32
_i32 = jnp.int32


@functools.lru_cache(maxsize=None)
def _static_plan(n, num_edges):
    rng = np.random.default_rng(0)
    perm = rng.permutation(n)
    num_mask = int(0.5 * n)
    mask_nodes = perm[:num_mask]
    perm_mask = rng.permutation(num_mask)
    num_noise = int(0.15 * num_mask)
    token_nodes = mask_nodes[perm_mask[: int(0.85 * num_mask)]]
    noise_nodes = mask_nodes[perm_mask[num_mask - num_noise:]]
    noise_chosen = rng.permutation(n)[:num_noise]
    kidx = np.nonzero(rng.random(num_edges) >= 0.5)[0].astype(np.int32)
    k = len(kidx)
    e_real = k + n
    ep = math.ceil(e_real / GRAN) * GRAN

    row_src = np.arange(n, dtype=np.int32)
    row_src[noise_nodes] = noise_chosen
    tflag = np.zeros((n, 1), np.float32)
    tflag[token_nodes] = 1.0
    mflag = np.zeros((n, 1), np.float32)
    mflag[mask_nodes] = 1.0
    eidx_p = np.zeros((ep,), np.int32)
    eidx_p[:k] = kidx
    return dict(
        k=k, e_real=e_real, ep=ep, num_mask=num_mask,
        kidx=kidx, row_src=row_src, tflag=tflag, mflag=mflag, eidx_p=eidx_p,
    )


def _sc_mesh():
    return plsc.VectorSubcoreMesh(core_axis_name="c", subcore_axis_name="s")


_SC_PARAMS = pltpu.CompilerParams(use_tc_tiling_on_sc=False)


@functools.lru_cache(maxsize=None)
def _make_gather(t_rows, r, m_pad):
    rpw = m_pad // NW
    nchunks = rpw // CH

    @functools.partial(
        pl.kernel,
        out_type=jax.ShapeDtypeStruct((m_pad, r), _f32),
        mesh=_sc_mesh(),
        compiler_params=_SC_PARAMS,
        scratch_types=[
            pltpu.VMEM((CH,), _i32),
            pltpu.VMEM((CH, r), _f32),
            pltpu.SemaphoreType.DMA,
        ],
    )
    def gather_k(table_hbm, idx_hbm, out_hbm, idx_v, rows_v, sem):
        c = lax.axis_index("c")
        s = lax.axis_index("s")
        wid = s * NC + c

        def body(i, carry):
            base = wid * rpw + i * CH
            pltpu.sync_copy(idx_hbm.at[pl.ds(base, CH)], idx_v)
            pltpu.async_copy(table_hbm.at[idx_v], rows_v, sem).wait()
            pltpu.sync_copy(rows_v, out_hbm.at[pl.ds(base, CH)])
            return carry

        lax.fori_loop(0, nchunks, body, 0)

    return gather_k


def _sc_gather(table, idx):
    m = idx.shape[0]
    m_pad = math.ceil(m / GRAN) * GRAN
    if m_pad > m:
        idx = jnp.concatenate([idx, jnp.zeros((m_pad - m,), _i32)])
    out = _make_gather(table.shape[0], table.shape[1], m_pad)(table, idx)
    return out[:m] if m_pad > m else out


@functools.lru_cache(maxsize=None)
def _make_scatter_add(n_out, r, m_pad):
    rpw = m_pad // NW
    nchunks = rpw // CH
    rpt = n_out // NS
    zch = math.gcd(rpt, CH)
    assert rpt * NS == n_out

    @functools.partial(
        pl.kernel,
        out_type=jax.ShapeDtypeStruct((NC, n_out, r), _f32),
        mesh=_sc_mesh(),
        compiler_params=_SC_PARAMS,
        scratch_types=[
            pltpu.VMEM((CH,), _i32),
            pltpu.VMEM((CH, r), _f32),
            pltpu.VMEM_SHARED((n_out, r), _f32),
            pltpu.SemaphoreType.DMA,
        ],
    )
    def scatter_k(vals_hbm, idx_hbm, out_hbm, idx_v, rows_v, acc, sem):
        c = lax.axis_index("c")
        s = lax.axis_index("s")
        wid = s * NC + c

        zero = jnp.zeros((16,), _f32)

        def zbody(i, carry):
            rows_v[i // (r // 16), pl.ds((i % (r // 16)) * 16, 16)] = zero
            return carry

        lax.fori_loop(0, CH * r // 16, zbody, 0)

        def zacc(j, carry):
            pltpu.sync_copy(rows_v.at[pl.ds(0, zch)],
                            acc.at[pl.ds(s * rpt + j * zch, zch)])
            return carry

        lax.fori_loop(0, rpt // zch, zacc, 0)
        plsc.subcore_barrier()

        def body(i, carry):
            base = wid * rpw + i * CH
            pltpu.sync_copy(idx_hbm.at[pl.ds(base, CH)], idx_v)
            pltpu.sync_copy(vals_hbm.at[pl.ds(base, CH)], rows_v)
            pltpu.sync_copy(rows_v, acc.at[idx_v], add=True)
            return carry

        lax.fori_loop(0, nchunks, body, 0)
        plsc.subcore_barrier()
        pltpu.sync_copy(acc.at[pl.ds(s * rpt, rpt)],
                        out_hbm.at[c].at[pl.ds(s * rpt, rpt)])

    return scatter_k


def _sc_scatter_add(vals, idx, n_out):
    m, r = vals.shape
    assert m % GRAN == 0
    return _make_scatter_add(n_out, r, m)(vals, idx)


def _full(shape):
    return pl.BlockSpec(shape, lambda i: tuple(0 for _ in shape))


def _rows(bshape):
    return pl.BlockSpec(bshape, lambda i: (i,) + tuple(0 for _ in bshape[1:]))


def _tc_mask_x(xg, tflag, mask_token):
    n = xg.shape[0]
    b = 1000

    def body(xg_ref, tf_ref, mt_ref, out_ref):
        tf = tf_ref[...]
        out_ref[...] = xg_ref[...] * (1.0 - tf) + tf * mt_ref[...]

    return pl.pallas_call(
        body,
        grid=(n // b,),
        in_specs=[_rows((b, D)), _rows((b, 1)), _full((1, D))],
        out_specs=_rows((b, D)),
        out_shape=jax.ShapeDtypeStruct((n, D), _f32),
    )(xg, tflag, mask_token)


def _tc_pre(h, wn, wni, wnj):
    n, d = h.shape
    b = 1000
    dh = wn.shape[1]
    de16 = wni.shape[1]

    def body(h_ref, wn_ref, wni_ref, wnj_ref, hn_ref, fi_ref, fj_ref):
        hb = h_ref[...]
        hn_ref[...] = jnp.dot(hb, wn_ref[...], preferred_element_type=_f32)
        fi_ref[...] = jnp.dot(hb, wni_ref[...], preferred_element_type=_f32)
        fj_ref[...] = jnp.dot(hb, wnj_ref[...], preferred_element_type=_f32)

    return pl.pallas_call(
        body,
        grid=(n // b,),
        in_specs=[_rows((b, d)), _full((d, dh)), _full((d, de16)), _full((d, de16))],
        out_specs=[_rows((b, dh)), _rows((b, de16)), _rows((b, de16))],
        out_shape=[
            jax.ShapeDtypeStruct((n, dh), _f32),
            jax.ShapeDtypeStruct((n, de16), _f32),
            jax.ShapeDtypeStruct((n, de16), _f32),
        ],
    )(h, wn, wni, wnj)


def _tc_matmul(a, w, zero_from=None):
    m, ka = a.shape
    kb, r = w.shape
    b = 4096 if m % 4096 == 0 else 1000

    def body(a_ref, w_ref, o_ref):
        o = jnp.dot(a_ref[...], w_ref[...], preferred_element_type=_f32)
        if zero_from is not None:
            row = lax.broadcasted_iota(_i32, (b, r), 0) + pl.program_id(0) * b
            o = jnp.where(row < zero_from, o, 0.0)
        o_ref[...] = o

    return pl.pallas_call(
        body,
        grid=(m // b,),
        in_specs=[_rows((b, ka)), _full((kb, r))],
        out_specs=_rows((b, r)),
        out_shape=jax.ShapeDtypeStruct((m, r), _f32),
    )(a, w)


def _tc_edge(fi_g, fj_g, fe, s_mat, heads, e_real, ln_s=None, ln_b=None):
    m = fi_g.shape[0]
    b = 4096
    with_en = ln_s is not None

    def body(*refs):
        if with_en:
            fi_ref, fj_ref, fe_ref, s_ref, lns_ref, lnb_ref, ex_ref, en_ref = refs
        else:
            fi_ref, fj_ref, fe_ref, s_ref, ex_ref = refs
        f = fi_ref[...] + fj_ref[...] + fe_ref[...]
        f = jnp.where(f > 0, f, 0.2 * f)
        sc = jnp.dot(f, s_ref[...], preferred_element_type=_f32)
        row = lax.broadcasted_iota(_i32, (b, 16), 0) + pl.program_id(0) * b
        col = lax.broadcasted_iota(_i32, (b, 16), 1)
        keep = jnp.logical_and(row < e_real, col < heads)
        ex_ref[...] = jnp.where(keep, jnp.exp(sc), 0.0)
        if with_en:
            r0 = jnp.maximum(f, 0.0)
            mu = jnp.mean(r0, axis=-1, keepdims=True)
            var = jnp.mean(r0 * r0, axis=-1, keepdims=True) - mu * mu
            en_ref[...] = (r0 - mu) * lax.rsqrt(var + 1e-5) * lns_ref[...] + lnb_ref[...]

    in_specs = [_rows((b, 16))] * 3 + [_full((16, 16))]
    out_specs = [_rows((b, 16))]
    out_shape = [jax.ShapeDtypeStruct((m, 16), _f32)]
    args = [fi_g, fj_g, fe, s_mat]
    if with_en:
        in_specs += [_full((1, 16)), _full((1, 16))]
        args += [ln_s.reshape(1, 16), ln_b.reshape(1, 16)]
        out_specs.append(_rows((b, 16)))
        out_shape.append(jax.ShapeDtypeStruct((m, 16), _f32))
    out = pl.pallas_call(
        body,
        grid=(m // b,),
        in_specs=in_specs,
        out_specs=out_specs if with_en else out_specs[0],
        out_shape=out_shape if with_en else out_shape[0],
    )(*args)
    return out if with_en else (out, None)


def _tc_recip_sum(dparts):
    n = dparts.shape[1]
    b = 1000

    def body(d0_ref, d1_ref, o_ref):
        o_ref[...] = 1.0 / (d0_ref[...] + d1_ref[...] + 1e-9)

    return pl.pallas_call(
        body,
        grid=(n // b,),
        in_specs=[_rows((b, 16)), _rows((b, 16))],
        out_specs=_rows((b, 16)),
        out_shape=jax.ShapeDtypeStruct((n, 16), _f32),
    )(dparts[0], dparts[1])


def _tc_msg(ex16, dgr, hn_g, x_mat):
    m = hn_g.shape[0]
    b = 2048

    def body(ex_ref, dg_ref, hn_ref, x_ref, o_ref):
        a = jnp.dot(ex_ref[...] * dg_ref[...], x_ref[...],
                    preferred_element_type=_f32)
        o_ref[...] = a * hn_ref[...]

    return pl.pallas_call(
        body,
        grid=(m // b,),
        in_specs=[_rows((b, 16)), _rows((b, 16)), _rows((b, D)), _full((16, D))],
        out_specs=_rows((b, D)),
        out_shape=jax.ShapeDtypeStruct((m, D), _f32),
    )(ex16, dgr, hn_g, x_mat)


def _tc_post(mparts, h, ln_s, ln_b):
    n = h.shape[0]
    b = 1000

    def body(p0_ref, p1_ref, h_ref, s_ref, bb_ref, o_ref):
        o = p0_ref[...] + p1_ref[...] + h_ref[...]
        o = jnp.maximum(o, 0.0)
        mu = jnp.mean(o, axis=-1, keepdims=True)
        var = jnp.mean(o * o, axis=-1, keepdims=True) - mu * mu
        o_ref[...] = (o - mu) * lax.rsqrt(var + 1e-5) * s_ref[...] + bb_ref[...]

    return pl.pallas_call(
        body,
        grid=(n // b,),
        in_specs=[_rows((b, D))] * 3 + [_full((1, D)), _full((1, D))],
        out_specs=_rows((b, D)),
        out_shape=jax.ShapeDtypeStruct((n, D), _f32),
    )(mparts[0], mparts[1], h, ln_s.reshape(1, D), ln_b.reshape(1, D))


def _tc_rep(h, w, mflag):
    n = h.shape[0]
    b = 1000

    def body(h_ref, w_ref, mf_ref, o_ref):
        o = jnp.dot(h_ref[...], w_ref[...], preferred_element_type=_f32)
        o_ref[...] = o * (1.0 - mf_ref[...])

    return pl.pallas_call(
        body,
        grid=(n // b,),
        in_specs=[_rows((b, D)), _full((D, D)), _rows((b, 1))],
        out_specs=_rows((b, D)),
        out_shape=jax.ShapeDtypeStruct((n, D), _f32),
    )(h, w, mflag)


def _tc_loss(mparts, rep, x, mflag, num_mask):
    n = x.shape[0]
    b = 1000

    def body(p0_ref, p1_ref, rep_ref, x_ref, mf_ref, o_ref):
        recon = p0_ref[...] + p1_ref[...] + rep_ref[...]
        xb = x_ref[...]
        nx = jnp.sqrt(jnp.sum(recon * recon, -1, keepdims=True)) + 1e-8
        ny = jnp.sqrt(jnp.sum(xb * xb, -1, keepdims=True)) + 1e-8
        cos = jnp.sum((recon / nx) * (xb / ny), -1, keepdims=True)
        v = (1.0 - cos) ** 3 * mf_ref[...]
        psum = jnp.sum(v, axis=0, keepdims=True)

        @pl.when(pl.program_id(0) == 0)
        def _():
            o_ref[...] = jnp.zeros((1, 1), _f32)

        o_ref[...] += psum

    out = pl.pallas_call(
        body,
        grid=(n // b,),
        in_specs=[_rows((b, D))] * 4 + [_rows((b, 1))],
        out_specs=pl.BlockSpec((1, 1), lambda i: (0, 0)),
        out_shape=jax.ShapeDtypeStruct((1, 1), _f32),
    )(mparts[0], mparts[1], rep, x, mflag)
    return out[0, 0] / np.float32(num_mask)


def _build_s(attn, de):
    attn_flat = attn.reshape(-1)
    rows = np.arange(16)
    s = jnp.zeros((16, 16), _f32).at[rows, rows // de].set(attn_flat)
    return s


@functools.lru_cache(maxsize=None)
def _build_x_mat(heads, dn):
    x = np.zeros((16, heads * dn), np.float32)
    for hh in range(heads):
        x[hh, hh * dn:(hh + 1) * dn] = 1.0
    return x


def _egat(h, ef, src_p, dst_p, lp, heads, dn, de, e_real, enc=True):
    hn, fi, fj = _tc_pre(h, lp["Wn"], lp["Wni"], lp["Wnj"])
    fe = _tc_matmul(ef, lp["We"], zero_from=None)
    fi_g = _sc_gather(fi, src_p)
    fj_g = _sc_gather(fj, dst_p)
    s_mat = _build_s(lp["attn"], de)
    if enc:
        ex16, en = _tc_edge(fi_g, fj_g, fe, s_mat, heads, e_real,
                            lp["ln_e_s"], lp["ln_e_b"])
    else:
        ex16, en = _tc_edge(fi_g, fj_g, fe, s_mat, heads, e_real)
    dparts = _sc_scatter_add(ex16, dst_p, N)
    dr16 = _tc_recip_sum(dparts)
    dgr = _sc_gather(dr16, dst_p)
    hn_g = _sc_gather(hn, src_p)
    x_mat = jnp.asarray(_build_x_mat(heads, dn))
    msg = _tc_msg(ex16, dgr, hn_g, x_mat)
    mparts = _sc_scatter_add(msg, dst_p, N)
    return mparts, en


def kernel(x, edge_index, e, params):
    n = x.shape[0]
    plan = _static_plan(n, edge_index.shape[1])
    k, e_real, ep = plan["k"], plan["e_real"], plan["ep"]

    row_src = jnp.asarray(plan["row_src"])
    tflag = jnp.asarray(plan["tflag"])
    mflag = jnp.asarray(plan["mflag"])
    eidx_p = jnp.asarray(plan["eidx_p"])
    kidx = jnp.asarray(plan["kidx"])

    loops = jnp.arange(n, dtype=_i32)
    padz = jnp.zeros((ep - e_real,), _i32)
    src_p = jnp.concatenate([jnp.take(edge_index[0], kidx), loops, padz])
    dst_p = jnp.concatenate([jnp.take(edge_index[1], kidx), loops, padz])

    xg = _sc_gather(x, row_src)
    h = _tc_mask_x(xg, tflag, params["mask_token"])

    e_g = _sc_gather(e, eidx_p)

    lp = params["enc0"]
    hn0, fi0, fj0 = _tc_pre(h, lp["Wn"], lp["Wni"], lp["Wnj"])
    fe0 = _tc_matmul(e_g, lp["We"], zero_from=k)
    fi0_g = _sc_gather(fi0, src_p)
    fj0_g = _sc_gather(fj0, dst_p)
    ex0, en0 = _tc_edge(fi0_g, fj0_g, fe0, _build_s(lp["attn"], DE_H), H, e_real,
                        lp["ln_e_s"], lp["ln_e_b"])
    d0 = _sc_scatter_add(ex0, dst_p, N)
    dr0 = _tc_recip_sum(d0)
    dg0 = _sc_gather(dr0, dst_p)
    hn0_g = _sc_gather(hn0, src_p)
    msg0 = _tc_msg(ex0, dg0, hn0_g, jnp.asarray(_build_x_mat(H, DN_H)))
    mp0 = _sc_scatter_add(msg0, dst_p, N)
    h = _tc_post(mp0, h, lp["ln_n_s"], lp["ln_n_b"])
    ef = en0

    lp = params["enc1"]
    mp1, en1 = _egat(h, ef, src_p, dst_p, lp, H, DN_H, DE_H, e_real, enc=True)
    h = _tc_post(mp1, h, lp["ln_n_s"], lp["ln_n_b"])
    ef = en1

    rep = _tc_rep(h, params["W_e2d"], mflag)
    rep_e = _tc_matmul(ef, params["W_e2d_e"])
    mpd, _ = _egat(rep, rep_e, src_p, dst_p, params["dec"], 1, D, DE,
                   e_real, enc=False)
    return _tc_loss(mpd, rep, x, mflag, plan["num_mask"])


# speedup vs baseline: 12.7073x; 1.1648x over previous
import functools
import math

import numpy as np
import jax
import jax.numpy as jnp
from jax import lax
from jax.experimental import pallas as pl
from jax.experimental.pallas import tpu as pltpu
from jax.experimental.pallas import tpu_sc as plsc

N = 10000
E = 320000
D = 128
DE = 16
H = 4
DN_H = 32
DE_H = 4

NC = 2
NS = 16
NW = NC * NS
CH = 128
GRAN = NW * CH

_f32 = jnp.float32
_i32 = jnp.int32


@functools.lru_cache(maxsize=None)
def _static_plan(n, num_edges):
    rng = np.random.default_rng(0)
    perm = rng.permutation(n)
    num_mask = int(0.5 * n)
    mask_nodes = perm[:num_mask]
    perm_mask = rng.permutation(num_mask)
    num_noise = int(0.15 * num_mask)
    token_nodes = mask_nodes[perm_mask[: int(0.85 * num_mask)]]
    noise_nodes = mask_nodes[perm_mask[num_mask - num_noise:]]
    noise_chosen = rng.permutation(n)[:num_noise]
    kidx = np.nonzero(rng.random(num_edges) >= 0.5)[0].astype(np.int32)
    k = len(kidx)
    e_real = k + n
    ep = math.ceil(e_real / GRAN) * GRAN

    row_src = np.arange(n, dtype=np.int32)
    row_src[noise_nodes] = noise_chosen
    tflag = np.zeros((n, 1), np.float32)
    tflag[token_nodes] = 1.0
    mflag = np.zeros((n, 1), np.float32)
    mflag[mask_nodes] = 1.0
    eidx_p = np.zeros((ep,), np.int32)
    eidx_p[:k] = kidx
    return dict(
        k=k, e_real=e_real, ep=ep, num_mask=num_mask,
        kidx=kidx, row_src=row_src, tflag=tflag, mflag=mflag, eidx_p=eidx_p,
    )


def _sc_mesh():
    return plsc.VectorSubcoreMesh(core_axis_name="c", subcore_axis_name="s")


_SC_PARAMS = pltpu.CompilerParams(use_tc_tiling_on_sc=False)


_STAGE_BYTES = 360_000
_W = 8


def _zero_fill(zbuf, rows, r):
    zero = jnp.zeros((16,), _f32)
    per = r // 16

    def zbody(i, carry):
        zbuf[i // per, pl.ds((i % per) * 16, 16)] = zero
        return carry

    lax.fori_loop(0, rows * per, zbody, 0)


@functools.lru_cache(maxsize=None)
def _make_gather(t_rows, r, m_pad):
    rpw = m_pad // NW
    nchunks = rpw // CH
    fits = rpw * r * 4 <= _STAGE_BYTES
    nb = nchunks if fits else 4

    @functools.partial(
        pl.kernel,
        out_type=jax.ShapeDtypeStruct((m_pad, r), _f32),
        mesh=_sc_mesh(),
        compiler_params=_SC_PARAMS,
        scratch_types=[
            pltpu.VMEM((nchunks, CH), _i32),
            pltpu.VMEM((nb * CH, r), _f32),
            pltpu.SemaphoreType.DMA,
            pltpu.SemaphoreType.DMA,
            pltpu.SemaphoreType.DMA,
        ],
    )
    def gather_k(table_hbm, idx_hbm, out_hbm, idx_v, rows_v, isem, gsem, osem):
        c = lax.axis_index("c")
        s = lax.axis_index("s")
        wid = s * NC + c
        pltpu.async_copy(idx_hbm.at[pl.ds(wid * nchunks, nchunks)],
                         idx_v, isem).wait()
        if fits:
            descs = [None] * nchunks
            for j in range(nchunks):
                if j >= _W:
                    descs[j - _W].wait()
                descs[j] = pltpu.async_copy(
                    table_hbm.at[idx_v.at[j]],
                    rows_v.at[pl.ds(j * CH, CH)], gsem)
            for j in range(max(0, nchunks - _W), nchunks):
                descs[j].wait()
            pltpu.sync_copy(rows_v, out_hbm.at[pl.ds(wid * rpw, rpw)])
        else:
            lag = nb - 1
            gd = [None] * nchunks
            od = [None] * nchunks
            for i in range(nchunks + lag):
                if i < nchunks:
                    b = i % nb
                    if i >= nb:
                        od[i - nb].wait()
                    gd[i] = pltpu.async_copy(
                        table_hbm.at[idx_v.at[i]],
                        rows_v.at[pl.ds(b * CH, CH)], gsem)
                j = i - lag
                if 0 <= j < nchunks:
                    gd[j].wait()
                    od[j] = pltpu.async_copy(
                        rows_v.at[pl.ds((j % nb) * CH, CH)],
                        out_hbm.at[pl.ds(wid * rpw + j * CH, CH)], osem)
            for j in range(max(0, nchunks - nb), nchunks):
                od[j].wait()

    return gather_k


def _sc_gather(table, idx):
    m = idx.shape[0]
    m_pad = math.ceil(m / GRAN) * GRAN
    if m_pad > m:
        idx = jnp.concatenate([idx, jnp.zeros((m_pad - m,), _i32)])
    out = _make_gather(table.shape[0], table.shape[1], m_pad)(
        table, idx.reshape(m_pad // CH, CH))
    return out[:m] if m_pad > m else out


@functools.lru_cache(maxsize=None)
def _make_scatter_add(n_out, r, m_pad):
    ch = 64 if r * 4 > 256 else CH
    rpw = m_pad // NW
    nchunks = rpw // ch
    rpt = n_out // NS
    assert rpt * NS == n_out
    zmax = 32 if r * 4 > 256 else CH
    zch = max(d for d in range(1, zmax + 1) if rpt % d == 0)
    fits = rpw * r * 4 <= _STAGE_BYTES
    nb = nchunks if fits else 4

    @functools.partial(
        pl.kernel,
        out_type=jax.ShapeDtypeStruct((NC, n_out, r), _f32),
        mesh=_sc_mesh(),
        compiler_params=_SC_PARAMS,
        scratch_types=[
            pltpu.VMEM((nchunks, ch), _i32),
            pltpu.VMEM((nb * ch, r), _f32),
            pltpu.VMEM((zch, r), _f32),
            pltpu.VMEM_SHARED((n_out, r), _f32),
            pltpu.SemaphoreType.DMA,
            pltpu.SemaphoreType.DMA,
            pltpu.SemaphoreType.DMA,
        ],
    )
    def scatter_k(vals_hbm, idx_hbm, out_hbm, idx_v, rows_v, zbuf, acc,
                  isem, vsem, ssem):
        c = lax.axis_index("c")
        s = lax.axis_index("s")
        wid = s * NC + c
        idesc = pltpu.async_copy(idx_hbm.at[pl.ds(wid * nchunks, nchunks)],
                                 idx_v, isem)
        _zero_fill(zbuf, zch, r)
        zd = [None] * (rpt // zch)
        for j in range(rpt // zch):
            zd[j] = pltpu.async_copy(
                zbuf, acc.at[pl.ds(s * rpt + j * zch, zch)], vsem)
        for d in zd:
            d.wait()
        idesc.wait()
        plsc.subcore_barrier()
        if fits:
            vdesc = pltpu.async_copy(
                vals_hbm.at[pl.ds(wid * rpw, rpw)], rows_v, vsem)
            vdesc.wait()
            sd = [None] * nchunks
            for j in range(nchunks):
                if j >= _W:
                    sd[j - _W].wait()
                sd[j] = pltpu.async_copy(
                    rows_v.at[pl.ds(j * ch, ch)],
                    acc.at[idx_v.at[j]], ssem, add=True)
            for j in range(max(0, nchunks - _W), nchunks):
                sd[j].wait()
        else:
            lag = nb - 1
            vd = [None] * nchunks
            sd = [None] * nchunks
            for i in range(nchunks + lag):
                if i < nchunks:
                    b = i % nb
                    if i >= nb:
                        sd[i - nb].wait()
                    vd[i] = pltpu.async_copy(
                        vals_hbm.at[pl.ds(wid * rpw + i * ch, ch)],
                        rows_v.at[pl.ds(b * ch, ch)], vsem)
                j = i - lag
                if 0 <= j < nchunks:
                    vd[j].wait()
                    sd[j] = pltpu.async_copy(
                        rows_v.at[pl.ds((j % nb) * ch, ch)],
                        acc.at[idx_v.at[j]], ssem, add=True)
            for j in range(max(0, nchunks - nb), nchunks):
                sd[j].wait()
        plsc.subcore_barrier()
        pltpu.sync_copy(acc.at[pl.ds(s * rpt, rpt)],
                        out_hbm.at[c].at[pl.ds(s * rpt, rpt)])

    return scatter_k


def _sc_scatter_add(vals, idx, n_out):
    m, r = vals.shape
    assert m % GRAN == 0
    ch = 64 if r * 4 > 256 else CH
    return _make_scatter_add(n_out, r, m)(vals, idx.reshape(m // ch, ch))


def _full(shape):
    return pl.BlockSpec(shape, lambda i: tuple(0 for _ in shape))


def _rows(bshape):
    return pl.BlockSpec(bshape, lambda i: (i,) + tuple(0 for _ in bshape[1:]))


def _tc_mask_x(xg, tflag, mask_token):
    n = xg.shape[0]
    b = 1000

    def body(xg_ref, tf_ref, mt_ref, out_ref):
        tf = tf_ref[...]
        out_ref[...] = xg_ref[...] * (1.0 - tf) + tf * mt_ref[...]

    return pl.pallas_call(
        body,
        grid=(n // b,),
        in_specs=[_rows((b, D)), _rows((b, 1)), _full((1, D))],
        out_specs=_rows((b, D)),
        out_shape=jax.ShapeDtypeStruct((n, D), _f32),
    )(xg, tflag, mask_token)


def _tc_pre(h, wn, wni, wnj):
    n, d = h.shape
    b = 1000
    dh = wn.shape[1]
    de16 = wni.shape[1]

    def body(h_ref, wn_ref, wni_ref, wnj_ref, hn_ref, fi_ref, fj_ref):
        hb = h_ref[...]
        hn_ref[...] = jnp.dot(hb, wn_ref[...], preferred_element_type=_f32)
        fi_ref[...] = jnp.dot(hb, wni_ref[...], preferred_element_type=_f32)
        fj_ref[...] = jnp.dot(hb, wnj_ref[...], preferred_element_type=_f32)

    return pl.pallas_call(
        body,
        grid=(n // b,),
        in_specs=[_rows((b, d)), _full((d, dh)), _full((d, de16)), _full((d, de16))],
        out_specs=[_rows((b, dh)), _rows((b, de16)), _rows((b, de16))],
        out_shape=[
            jax.ShapeDtypeStruct((n, dh), _f32),
            jax.ShapeDtypeStruct((n, de16), _f32),
            jax.ShapeDtypeStruct((n, de16), _f32),
        ],
    )(h, wn, wni, wnj)


def _tc_matmul(a, w, zero_from=None):
    m, ka = a.shape
    kb, r = w.shape
    b = 4096 if m % 4096 == 0 else 1000

    def body(a_ref, w_ref, o_ref):
        o = jnp.dot(a_ref[...], w_ref[...], preferred_element_type=_f32)
        if zero_from is not None:
            row = lax.broadcasted_iota(_i32, (b, r), 0) + pl.program_id(0) * b
            o = jnp.where(row < zero_from, o, 0.0)
        o_ref[...] = o

    return pl.pallas_call(
        body,
        grid=(m // b,),
        in_specs=[_rows((b, ka)), _full((kb, r))],
        out_specs=_rows((b, r)),
        out_shape=jax.ShapeDtypeStruct((m, r), _f32),
    )(a, w)


def _tc_edge(fi_g, fj_g, fe, s_mat, heads, e_real, ln_s=None, ln_b=None):
    m = fi_g.shape[0]
    b = 4096
    with_en = ln_s is not None

    def body(*refs):
        if with_en:
            fi_ref, fj_ref, fe_ref, s_ref, lns_ref, lnb_ref, ex_ref, en_ref = refs
        else:
            fi_ref, fj_ref, fe_ref, s_ref, ex_ref = refs
        f = fi_ref[...] + fj_ref[...] + fe_ref[...]
        f = jnp.where(f > 0, f, 0.2 * f)
        sc = jnp.dot(f, s_ref[...], preferred_element_type=_f32)
        row = lax.broadcasted_iota(_i32, (b, 16), 0) + pl.program_id(0) * b
        col = lax.broadcasted_iota(_i32, (b, 16), 1)
        keep = jnp.logical_and(row < e_real, col < heads)
        ex_ref[...] = jnp.where(keep, jnp.exp(sc), 0.0)
        if with_en:
            r0 = jnp.maximum(f, 0.0)
            mu = jnp.mean(r0, axis=-1, keepdims=True)
            var = jnp.mean(r0 * r0, axis=-1, keepdims=True) - mu * mu
            en_ref[...] = (r0 - mu) * lax.rsqrt(var + 1e-5) * lns_ref[...] + lnb_ref[...]

    in_specs = [_rows((b, 16))] * 3 + [_full((16, 16))]
    out_specs = [_rows((b, 16))]
    out_shape = [jax.ShapeDtypeStruct((m, 16), _f32)]
    args = [fi_g, fj_g, fe, s_mat]
    if with_en:
        in_specs += [_full((1, 16)), _full((1, 16))]
        args += [ln_s.reshape(1, 16), ln_b.reshape(1, 16)]
        out_specs.append(_rows((b, 16)))
        out_shape.append(jax.ShapeDtypeStruct((m, 16), _f32))
    out = pl.pallas_call(
        body,
        grid=(m // b,),
        in_specs=in_specs,
        out_specs=out_specs if with_en else out_specs[0],
        out_shape=out_shape if with_en else out_shape[0],
    )(*args)
    return out if with_en else (out, None)


def _tc_recip_sum(dparts):
    n = dparts.shape[1]
    b = 1000

    def body(d0_ref, d1_ref, o_ref):
        o_ref[...] = 1.0 / (d0_ref[...] + d1_ref[...] + 1e-9)

    return pl.pallas_call(
        body,
        grid=(n // b,),
        in_specs=[_rows((b, 16)), _rows((b, 16))],
        out_specs=_rows((b, 16)),
        out_shape=jax.ShapeDtypeStruct((n, 16), _f32),
    )(dparts[0], dparts[1])


def _tc_msg(ex16, dgr, hn_g, x_mat):
    m = hn_g.shape[0]
    b = 2048

    def body(ex_ref, dg_ref, hn_ref, x_ref, o_ref):
        a = jnp.dot(ex_ref[...] * dg_ref[...], x_ref[...],
                    preferred_element_type=_f32)
        o_ref[...] = a * hn_ref[...]

    return pl.pallas_call(
        body,
        grid=(m // b,),
        in_specs=[_rows((b, 16)), _rows((b, 16)), _rows((b, D)), _full((16, D))],
        out_specs=_rows((b, D)),
        out_shape=jax.ShapeDtypeStruct((m, D), _f32),
    )(ex16, dgr, hn_g, x_mat)


def _tc_post(mparts, h, ln_s, ln_b):
    n = h.shape[0]
    b = 1000

    def body(p0_ref, p1_ref, h_ref, s_ref, bb_ref, o_ref):
        o = p0_ref[...] + p1_ref[...] + h_ref[...]
        o = jnp.maximum(o, 0.0)
        mu = jnp.mean(o, axis=-1, keepdims=True)
        var = jnp.mean(o * o, axis=-1, keepdims=True) - mu * mu
        o_ref[...] = (o - mu) * lax.rsqrt(var + 1e-5) * s_ref[...] + bb_ref[...]

    return pl.pallas_call(
        body,
        grid=(n // b,),
        in_specs=[_rows((b, D))] * 3 + [_full((1, D)), _full((1, D))],
        out_specs=_rows((b, D)),
        out_shape=jax.ShapeDtypeStruct((n, D), _f32),
    )(mparts[0], mparts[1], h, ln_s.reshape(1, D), ln_b.reshape(1, D))


def _tc_rep(h, w, mflag):
    n = h.shape[0]
    b = 1000

    def body(h_ref, w_ref, mf_ref, o_ref):
        o = jnp.dot(h_ref[...], w_ref[...], preferred_element_type=_f32)
        o_ref[...] = o * (1.0 - mf_ref[...])

    return pl.pallas_call(
        body,
        grid=(n // b,),
        in_specs=[_rows((b, D)), _full((D, D)), _rows((b, 1))],
        out_specs=_rows((b, D)),
        out_shape=jax.ShapeDtypeStruct((n, D), _f32),
    )(h, w, mflag)


def _tc_loss(mparts, rep, x, mflag, num_mask):
    n = x.shape[0]
    b = 1000

    def body(p0_ref, p1_ref, rep_ref, x_ref, mf_ref, o_ref):
        recon = p0_ref[...] + p1_ref[...] + rep_ref[...]
        xb = x_ref[...]
        nx = jnp.sqrt(jnp.sum(recon * recon, -1, keepdims=True)) + 1e-8
        ny = jnp.sqrt(jnp.sum(xb * xb, -1, keepdims=True)) + 1e-8
        cos = jnp.sum((recon / nx) * (xb / ny), -1, keepdims=True)
        v = (1.0 - cos) ** 3 * mf_ref[...]
        psum = jnp.sum(v, axis=0, keepdims=True)

        @pl.when(pl.program_id(0) == 0)
        def _():
            o_ref[...] = jnp.zeros((1, 1), _f32)

        o_ref[...] += psum

    out = pl.pallas_call(
        body,
        grid=(n // b,),
        in_specs=[_rows((b, D))] * 4 + [_rows((b, 1))],
        out_specs=pl.BlockSpec((1, 1), lambda i: (0, 0)),
        out_shape=jax.ShapeDtypeStruct((1, 1), _f32),
    )(mparts[0], mparts[1], rep, x, mflag)
    return out[0, 0] / np.float32(num_mask)


def _build_s(attn, de):
    attn_flat = attn.reshape(-1)
    rows = np.arange(16)
    s = jnp.zeros((16, 16), _f32).at[rows, rows // de].set(attn_flat)
    return s


@functools.lru_cache(maxsize=None)
def _build_x_mat(heads, dn):
    x = np.zeros((16, heads * dn), np.float32)
    for hh in range(heads):
        x[hh, hh * dn:(hh + 1) * dn] = 1.0
    return x


def _egat(h, ef, src_p, dst_p, lp, heads, dn, de, e_real, enc=True):
    hn, fi, fj = _tc_pre(h, lp["Wn"], lp["Wni"], lp["Wnj"])
    fe = _tc_matmul(ef, lp["We"], zero_from=None)
    fi_g = _sc_gather(fi, src_p)
    fj_g = _sc_gather(fj, dst_p)
    s_mat = _build_s(lp["attn"], de)
    if enc:
        ex16, en = _tc_edge(fi_g, fj_g, fe, s_mat, heads, e_real,
                            lp["ln_e_s"], lp["ln_e_b"])
    else:
        ex16, en = _tc_edge(fi_g, fj_g, fe, s_mat, heads, e_real)
    dparts = _sc_scatter_add(ex16, dst_p, N)
    dr16 = _tc_recip_sum(dparts)
    dgr = _sc_gather(dr16, dst_p)
    hn_g = _sc_gather(hn, src_p)
    x_mat = jnp.asarray(_build_x_mat(heads, dn))
    msg = _tc_msg(ex16, dgr, hn_g, x_mat)
    mparts = _sc_scatter_add(msg, dst_p, N)
    return mparts, en


def kernel(x, edge_index, e, params):
    n = x.shape[0]
    plan = _static_plan(n, edge_index.shape[1])
    k, e_real, ep = plan["k"], plan["e_real"], plan["ep"]

    row_src = jnp.asarray(plan["row_src"])
    tflag = jnp.asarray(plan["tflag"])
    mflag = jnp.asarray(plan["mflag"])
    eidx_p = jnp.asarray(plan["eidx_p"])
    kidx = jnp.asarray(plan["kidx"])

    loops = jnp.arange(n, dtype=_i32)
    padz = jnp.zeros((ep - e_real,), _i32)
    src_p = jnp.concatenate([jnp.take(edge_index[0], kidx), loops, padz])
    dst_p = jnp.concatenate([jnp.take(edge_index[1], kidx), loops, padz])

    xg = _sc_gather(x, row_src)
    h = _tc_mask_x(xg, tflag, params["mask_token"])

    e_g = _sc_gather(e, eidx_p)

    lp = params["enc0"]
    hn0, fi0, fj0 = _tc_pre(h, lp["Wn"], lp["Wni"], lp["Wnj"])
    fe0 = _tc_matmul(e_g, lp["We"], zero_from=k)
    fi0_g = _sc_gather(fi0, src_p)
    fj0_g = _sc_gather(fj0, dst_p)
    ex0, en0 = _tc_edge(fi0_g, fj0_g, fe0, _build_s(lp["attn"], DE_H), H, e_real,
                        lp["ln_e_s"], lp["ln_e_b"])
    d0 = _sc_scatter_add(ex0, dst_p, N)
    dr0 = _tc_recip_sum(d0)
    dg0 = _sc_gather(dr0, dst_p)
    hn0_g = _sc_gather(hn0, src_p)
    msg0 = _tc_msg(ex0, dg0, hn0_g, jnp.asarray(_build_x_mat(H, DN_H)))
    mp0 = _sc_scatter_add(msg0, dst_p, N)
    h = _tc_post(mp0, h, lp["ln_n_s"], lp["ln_n_b"])
    ef = en0

    lp = params["enc1"]
    mp1, en1 = _egat(h, ef, src_p, dst_p, lp, H, DN_H, DE_H, e_real, enc=True)
    h = _tc_post(mp1, h, lp["ln_n_s"], lp["ln_n_b"])
    ef = en1

    rep = _tc_rep(h, params["W_e2d"], mflag)
    rep_e = _tc_matmul(ef, params["W_e2d_e"])
    mpd, _ = _egat(rep, rep_e, src_p, dst_p, params["dec"], 1, D, DE,
                   e_real, enc=False)
    return _tc_loss(mpd, rep, x, mflag, plan["num_mask"])


# speedup vs baseline: 17.0209x; 1.3395x over previous
import functools
import math

import numpy as np
import jax
import jax.numpy as jnp
from jax import lax
from jax.experimental import pallas as pl
from jax.experimental.pallas import tpu as pltpu
from jax.experimental.pallas import tpu_sc as plsc

N = 10000
E = 320000
D = 128
DE = 16
H = 4
DN_H = 32
DE_H = 4

NC = 2
NS = 16
NW = NC * NS
CH = 128
GRAN = NW * CH

_f32 = jnp.float32
_i32 = jnp.int32


@functools.lru_cache(maxsize=None)
def _static_plan(n, num_edges):
    rng = np.random.default_rng(0)
    perm = rng.permutation(n)
    num_mask = int(0.5 * n)
    mask_nodes = perm[:num_mask]
    perm_mask = rng.permutation(num_mask)
    num_noise = int(0.15 * num_mask)
    token_nodes = mask_nodes[perm_mask[: int(0.85 * num_mask)]]
    noise_nodes = mask_nodes[perm_mask[num_mask - num_noise:]]
    noise_chosen = rng.permutation(n)[:num_noise]
    kidx = np.nonzero(rng.random(num_edges) >= 0.5)[0].astype(np.int32)
    k = len(kidx)
    e_real = k + n
    ep = math.ceil(e_real / GRAN) * GRAN

    row_src = np.arange(n, dtype=np.int32)
    row_src[noise_nodes] = noise_chosen
    tflag = np.zeros((n, 1), np.float32)
    tflag[token_nodes] = 1.0
    mflag = np.zeros((n, 1), np.float32)
    mflag[mask_nodes] = 1.0
    eidx_p = np.zeros((ep,), np.int32)
    eidx_p[:k] = kidx
    n_pad = math.ceil(n / GRAN) * GRAN
    rowsrc_p = np.zeros((n_pad,), np.int32)
    rowsrc_p[:n] = row_src
    pairs_idx = np.full((ep,), num_edges, np.int32)
    pairs_idx[:k] = kidx
    pairs_idx[k:e_real] = num_edges + np.arange(n, dtype=np.int32)
    loop_pairs = np.stack([np.arange(n, dtype=np.int32)] * 2, axis=1)
    return dict(
        k=k, e_real=e_real, ep=ep, num_mask=num_mask,
        tflag=tflag, mflag=mflag,
        eidx2d=eidx_p.reshape(-1, CH),
        rowsrc2d=rowsrc_p.reshape(-1, CH),
        pairs_idx2d=pairs_idx.reshape(-1, CH),
        loop_pairs=loop_pairs,
    )


def _sc_mesh():
    return plsc.VectorSubcoreMesh(core_axis_name="c", subcore_axis_name="s")


_SC_PARAMS = pltpu.CompilerParams(use_tc_tiling_on_sc=False)


_STAGE_BYTES = 360_000
_W = 8


def _zero_fill(zbuf, rows, r):
    zero = jnp.zeros((16,), _f32)
    per = r // 16

    def zbody(i, carry):
        zbuf[i // per, pl.ds((i % per) * 16, 16)] = zero
        return carry

    lax.fori_loop(0, rows * per, zbody, 0)


@functools.lru_cache(maxsize=None)
def _make_gather(t_rows, r, m_pad, dtype=_f32):
    rpw = m_pad // NW
    nchunks = rpw // CH
    fits = rpw * r * 4 <= _STAGE_BYTES
    nb = nchunks if fits else 4

    @functools.partial(
        pl.kernel,
        out_type=jax.ShapeDtypeStruct((m_pad, r), dtype),
        mesh=_sc_mesh(),
        compiler_params=_SC_PARAMS,
        scratch_types=[
            pltpu.VMEM((nchunks, CH), _i32),
            pltpu.VMEM((nb * CH, r), dtype),
            pltpu.SemaphoreType.DMA,
            pltpu.SemaphoreType.DMA,
            pltpu.SemaphoreType.DMA,
        ],
    )
    def gather_k(table_hbm, idx_hbm, out_hbm, idx_v, rows_v, isem, gsem, osem):
        c = lax.axis_index("c")
        s = lax.axis_index("s")
        wid = s * NC + c
        pltpu.async_copy(idx_hbm.at[pl.ds(wid * nchunks, nchunks)],
                         idx_v, isem).wait()
        if fits:
            descs = [None] * nchunks
            for j in range(nchunks):
                if j >= _W:
                    descs[j - _W].wait()
                descs[j] = pltpu.async_copy(
                    table_hbm.at[idx_v.at[j]],
                    rows_v.at[pl.ds(j * CH, CH)], gsem)
            for j in range(max(0, nchunks - _W), nchunks):
                descs[j].wait()
            pltpu.sync_copy(rows_v, out_hbm.at[pl.ds(wid * rpw, rpw)])
        else:
            lag = nb - 1
            gd = [None] * nchunks
            od = [None] * nchunks
            for i in range(nchunks + lag):
                if i < nchunks:
                    b = i % nb
                    if i >= nb:
                        od[i - nb].wait()
                    gd[i] = pltpu.async_copy(
                        table_hbm.at[idx_v.at[i]],
                        rows_v.at[pl.ds(b * CH, CH)], gsem)
                j = i - lag
                if 0 <= j < nchunks:
                    gd[j].wait()
                    od[j] = pltpu.async_copy(
                        rows_v.at[pl.ds((j % nb) * CH, CH)],
                        out_hbm.at[pl.ds(wid * rpw + j * CH, CH)], osem)
            for j in range(max(0, nchunks - nb), nchunks):
                od[j].wait()

    return gather_k


def _sc_gather(table, idx2d):
    m_pad = idx2d.shape[0] * CH
    return _make_gather(table.shape[0], table.shape[1], m_pad,
                        table.dtype)(table, idx2d)


@functools.lru_cache(maxsize=None)
def _make_scatter_add(n_out, r, m_pad):
    ch = 64 if r * 4 > 256 else CH
    rpw = m_pad // NW
    nchunks = rpw // ch
    rpt = n_out // NS
    assert rpt * NS == n_out
    zmax = 32 if r * 4 > 256 else CH
    zch = max(d for d in range(1, zmax + 1) if rpt % d == 0)
    fits = rpw * r * 4 <= _STAGE_BYTES
    nb = nchunks if fits else 4

    @functools.partial(
        pl.kernel,
        out_type=jax.ShapeDtypeStruct((NC, n_out, r), _f32),
        mesh=_sc_mesh(),
        compiler_params=_SC_PARAMS,
        scratch_types=[
            pltpu.VMEM((nchunks, ch), _i32),
            pltpu.VMEM((nb * ch, r), _f32),
            pltpu.VMEM((zch, r), _f32),
            pltpu.VMEM_SHARED((n_out, r), _f32),
            pltpu.SemaphoreType.DMA,
            pltpu.SemaphoreType.DMA,
            pltpu.SemaphoreType.DMA,
        ],
    )
    def scatter_k(vals_hbm, idx_hbm, out_hbm, idx_v, rows_v, zbuf, acc,
                  isem, vsem, ssem):
        c = lax.axis_index("c")
        s = lax.axis_index("s")
        wid = s * NC + c
        idesc = pltpu.async_copy(idx_hbm.at[pl.ds(wid * nchunks, nchunks)],
                                 idx_v, isem)
        _zero_fill(zbuf, zch, r)
        zd = [None] * (rpt // zch)
        for j in range(rpt // zch):
            zd[j] = pltpu.async_copy(
                zbuf, acc.at[pl.ds(s * rpt + j * zch, zch)], vsem)
        for d in zd:
            d.wait()
        idesc.wait()
        plsc.subcore_barrier()
        if fits:
            vdesc = pltpu.async_copy(
                vals_hbm.at[pl.ds(wid * rpw, rpw)], rows_v, vsem)
            vdesc.wait()
            sd = [None] * nchunks
            for j in range(nchunks):
                if j >= _W:
                    sd[j - _W].wait()
                sd[j] = pltpu.async_copy(
                    rows_v.at[pl.ds(j * ch, ch)],
                    acc.at[idx_v.at[j]], ssem, add=True)
            for j in range(max(0, nchunks - _W), nchunks):
                sd[j].wait()
        else:
            lag = nb - 1
            vd = [None] * nchunks
            sd = [None] * nchunks
            for i in range(nchunks + lag):
                if i < nchunks:
                    b = i % nb
                    if i >= nb:
                        sd[i - nb].wait()
                    vd[i] = pltpu.async_copy(
                        vals_hbm.at[pl.ds(wid * rpw + i * ch, ch)],
                        rows_v.at[pl.ds(b * ch, ch)], vsem)
                j = i - lag
                if 0 <= j < nchunks:
                    vd[j].wait()
                    sd[j] = pltpu.async_copy(
                        rows_v.at[pl.ds((j % nb) * ch, ch)],
                        acc.at[idx_v.at[j]], ssem, add=True)
            for j in range(max(0, nchunks - nb), nchunks):
                sd[j].wait()
        plsc.subcore_barrier()
        pltpu.sync_copy(acc.at[pl.ds(s * rpt, rpt)],
                        out_hbm.at[c].at[pl.ds(s * rpt, rpt)])

    return scatter_k


def _sc_scatter_add(vals, idx2d, n_out):
    m, r = vals.shape
    assert m % GRAN == 0
    ch = 64 if r * 4 > 256 else CH
    assert idx2d.shape == (m // ch, ch)
    return _make_scatter_add(n_out, r, m)(vals, idx2d)


def _full(shape):
    return pl.BlockSpec(shape, lambda i: tuple(0 for _ in shape))


def _rows(bshape):
    return pl.BlockSpec(bshape, lambda i: (i,) + tuple(0 for _ in bshape[1:]))


def _tc_mask_x(xg, tflag, mask_token):
    n = tflag.shape[0]
    b = 1000

    def body(xg_ref, tf_ref, mt_ref, out_ref):
        tf = tf_ref[...]
        out_ref[...] = xg_ref[...] * (1.0 - tf) + tf * mt_ref[...]

    return pl.pallas_call(
        body,
        grid=(n // b,),
        in_specs=[_rows((b, D)), _rows((b, 1)), _full((1, D))],
        out_specs=_rows((b, D)),
        out_shape=jax.ShapeDtypeStruct((n, D), _f32),
    )(xg, tflag, mask_token)


def _tc_pre(h, wn, wni, wnj):
    n, d = h.shape
    b = 1000
    dh = wn.shape[1]
    de16 = wni.shape[1]

    def body(h_ref, wn_ref, wni_ref, wnj_ref, hn_ref, fi_ref, fj_ref):
        hb = h_ref[...]
        hn_ref[...] = jnp.dot(hb, wn_ref[...], preferred_element_type=_f32)
        fi_ref[...] = jnp.dot(hb, wni_ref[...], preferred_element_type=_f32)
        fj_ref[...] = jnp.dot(hb, wnj_ref[...], preferred_element_type=_f32)

    return pl.pallas_call(
        body,
        grid=(n // b,),
        in_specs=[_rows((b, d)), _full((d, dh)), _full((d, de16)), _full((d, de16))],
        out_specs=[_rows((b, dh)), _rows((b, de16)), _rows((b, de16))],
        out_shape=[
            jax.ShapeDtypeStruct((n, dh), _f32),
            jax.ShapeDtypeStruct((n, de16), _f32),
            jax.ShapeDtypeStruct((n, de16), _f32),
        ],
    )(h, wn, wni, wnj)


def _tc_matmul(a, w, zero_from=None):
    m, ka = a.shape
    kb, r = w.shape
    b = 4096 if m % 4096 == 0 else 1000

    def body(a_ref, w_ref, o_ref):
        o = jnp.dot(a_ref[...], w_ref[...], preferred_element_type=_f32)
        if zero_from is not None:
            row = lax.broadcasted_iota(_i32, (b, r), 0) + pl.program_id(0) * b
            o = jnp.where(row < zero_from, o, 0.0)
        o_ref[...] = o

    return pl.pallas_call(
        body,
        grid=(m // b,),
        in_specs=[_rows((b, ka)), _full((kb, r))],
        out_specs=_rows((b, r)),
        out_shape=jax.ShapeDtypeStruct((m, r), _f32),
    )(a, w)


def _tc_edge(fi_g, fj_g, fe, s_mat, heads, e_real, ln_s=None, ln_b=None):
    m = fi_g.shape[0]
    b = 4096
    with_en = ln_s is not None

    def body(*refs):
        if with_en:
            fi_ref, fj_ref, fe_ref, s_ref, lns_ref, lnb_ref, ex_ref, en_ref = refs
        else:
            fi_ref, fj_ref, fe_ref, s_ref, ex_ref = refs
        f = fi_ref[...] + fj_ref[...] + fe_ref[...]
        f = jnp.where(f > 0, f, 0.2 * f)
        sc = jnp.dot(f, s_ref[...], preferred_element_type=_f32)
        row = lax.broadcasted_iota(_i32, (b, 16), 0) + pl.program_id(0) * b
        col = lax.broadcasted_iota(_i32, (b, 16), 1)
        keep = jnp.logical_and(row < e_real, col < heads)
        ex_ref[...] = jnp.where(keep, jnp.exp(sc), 0.0)
        if with_en:
            r0 = jnp.maximum(f, 0.0)
            mu = jnp.mean(r0, axis=-1, keepdims=True)
            var = jnp.mean(r0 * r0, axis=-1, keepdims=True) - mu * mu
            en_ref[...] = (r0 - mu) * lax.rsqrt(var + 1e-5) * lns_ref[...] + lnb_ref[...]

    in_specs = [_rows((b, 16))] * 3 + [_full((16, 16))]
    out_specs = [_rows((b, 16))]
    out_shape = [jax.ShapeDtypeStruct((m, 16), _f32)]
    args = [fi_g, fj_g, fe, s_mat]
    if with_en:
        in_specs += [_full((1, 16)), _full((1, 16))]
        args += [ln_s.reshape(1, 16), ln_b.reshape(1, 16)]
        out_specs.append(_rows((b, 16)))
        out_shape.append(jax.ShapeDtypeStruct((m, 16), _f32))
    out = pl.pallas_call(
        body,
        grid=(m // b,),
        in_specs=in_specs,
        out_specs=out_specs if with_en else out_specs[0],
        out_shape=out_shape if with_en else out_shape[0],
    )(*args)
    return out if with_en else (out, None)


def _tc_recip_sum(dparts):
    n = dparts.shape[1]
    b = 1000

    def body(d0_ref, d1_ref, o_ref):
        o_ref[...] = 1.0 / (d0_ref[...] + d1_ref[...] + 1e-9)

    return pl.pallas_call(
        body,
        grid=(n // b,),
        in_specs=[_rows((b, 16)), _rows((b, 16))],
        out_specs=_rows((b, 16)),
        out_shape=jax.ShapeDtypeStruct((n, 16), _f32),
    )(dparts[0], dparts[1])


def _tc_msg(ex16, dgr, hn_g, x_mat):
    m = hn_g.shape[0]
    b = 2048

    def body(ex_ref, dg_ref, hn_ref, x_ref, o_ref):
        a = jnp.dot(ex_ref[...] * dg_ref[...], x_ref[...],
                    preferred_element_type=_f32)
        o_ref[...] = a * hn_ref[...]

    return pl.pallas_call(
        body,
        grid=(m // b,),
        in_specs=[_rows((b, 16)), _rows((b, 16)), _rows((b, D)), _full((16, D))],
        out_specs=_rows((b, D)),
        out_shape=jax.ShapeDtypeStruct((m, D), _f32),
    )(ex16, dgr, hn_g, x_mat)


def _tc_post(mparts, h, ln_s, ln_b):
    n = h.shape[0]
    b = 1000

    def body(p0_ref, p1_ref, h_ref, s_ref, bb_ref, o_ref):
        o = p0_ref[...] + p1_ref[...] + h_ref[...]
        o = jnp.maximum(o, 0.0)
        mu = jnp.mean(o, axis=-1, keepdims=True)
        var = jnp.mean(o * o, axis=-1, keepdims=True) - mu * mu
        o_ref[...] = (o - mu) * lax.rsqrt(var + 1e-5) * s_ref[...] + bb_ref[...]

    return pl.pallas_call(
        body,
        grid=(n // b,),
        in_specs=[_rows((b, D))] * 3 + [_full((1, D)), _full((1, D))],
        out_specs=_rows((b, D)),
        out_shape=jax.ShapeDtypeStruct((n, D), _f32),
    )(mparts[0], mparts[1], h, ln_s.reshape(1, D), ln_b.reshape(1, D))


def _tc_rep(h, w, mflag):
    n = h.shape[0]
    b = 1000

    def body(h_ref, w_ref, mf_ref, o_ref):
        o = jnp.dot(h_ref[...], w_ref[...], preferred_element_type=_f32)
        o_ref[...] = o * (1.0 - mf_ref[...])

    return pl.pallas_call(
        body,
        grid=(n // b,),
        in_specs=[_rows((b, D)), _full((D, D)), _rows((b, 1))],
        out_specs=_rows((b, D)),
        out_shape=jax.ShapeDtypeStruct((n, D), _f32),
    )(h, w, mflag)


def _tc_loss(mparts, rep, x, mflag, num_mask):
    n = x.shape[0]
    b = 1000

    def body(p0_ref, p1_ref, rep_ref, x_ref, mf_ref, o_ref):
        recon = p0_ref[...] + p1_ref[...] + rep_ref[...]
        xb = x_ref[...]
        nx = jnp.sqrt(jnp.sum(recon * recon, -1, keepdims=True)) + 1e-8
        ny = jnp.sqrt(jnp.sum(xb * xb, -1, keepdims=True)) + 1e-8
        cos = jnp.sum((recon / nx) * (xb / ny), -1, keepdims=True)
        v = (1.0 - cos) ** 3 * mf_ref[...]
        psum = jnp.sum(v, axis=0, keepdims=True)

        @pl.when(pl.program_id(0) == 0)
        def _():
            o_ref[...] = jnp.zeros((1, 1), _f32)

        o_ref[...] += psum

    out = pl.pallas_call(
        body,
        grid=(n // b,),
        in_specs=[_rows((b, D))] * 4 + [_rows((b, 1))],
        out_specs=pl.BlockSpec((1, 1), lambda i: (0, 0)),
        out_shape=jax.ShapeDtypeStruct((1, 1), _f32),
    )(mparts[0], mparts[1], rep, x, mflag)
    return out[0, 0] / np.float32(num_mask)


def _build_s(attn, de):
    attn_flat = attn.reshape(-1)
    rows = np.arange(16)
    s = jnp.zeros((16, 16), _f32).at[rows, rows // de].set(attn_flat)
    return s


@functools.lru_cache(maxsize=None)
def _build_x_mat(heads, dn):
    x = np.zeros((16, heads * dn), np.float32)
    for hh in range(heads):
        x[hh, hh * dn:(hh + 1) * dn] = 1.0
    return x


def _egat(h, ef, src128, dst128, dst64, lp, heads, dn, de, e_real, enc=True):
    hn, fi, fj = _tc_pre(h, lp["Wn"], lp["Wni"], lp["Wnj"])
    fe = _tc_matmul(ef, lp["We"], zero_from=None)
    fi_g = _sc_gather(fi, src128)
    fj_g = _sc_gather(fj, dst128)
    s_mat = _build_s(lp["attn"], de)
    if enc:
        ex16, en = _tc_edge(fi_g, fj_g, fe, s_mat, heads, e_real,
                            lp["ln_e_s"], lp["ln_e_b"])
    else:
        ex16, en = _tc_edge(fi_g, fj_g, fe, s_mat, heads, e_real)
    dparts = _sc_scatter_add(ex16, dst128, N)
    dr16 = _tc_recip_sum(dparts)
    dgr = _sc_gather(dr16, dst128)
    hn_g = _sc_gather(hn, src128)
    x_mat = jnp.asarray(_build_x_mat(heads, dn))
    msg = _tc_msg(ex16, dgr, hn_g, x_mat)
    mparts = _sc_scatter_add(msg, dst64, N)
    return mparts, en


def kernel(x, edge_index, e, params):
    n = x.shape[0]
    plan = _static_plan(n, edge_index.shape[1])
    k, e_real, ep = plan["k"], plan["e_real"], plan["ep"]

    tflag = jnp.asarray(plan["tflag"])
    mflag = jnp.asarray(plan["mflag"])

    pairs = jnp.concatenate(
        [edge_index.T, jnp.asarray(plan["loop_pairs"])], axis=0)
    table16 = jnp.pad(pairs, ((0, 0), (0, 14)))
    pairs_p = _sc_gather(table16, jnp.asarray(plan["pairs_idx2d"]))
    src_p = pairs_p[:, 0]
    dst_p = pairs_p[:, 1]
    src128 = src_p.reshape(-1, CH)
    dst128 = dst_p.reshape(-1, CH)
    dst64 = dst_p.reshape(-1, 64)

    xg = _sc_gather(x, jnp.asarray(plan["rowsrc2d"]))
    h = _tc_mask_x(xg, tflag, params["mask_token"])

    e_g = _sc_gather(e, jnp.asarray(plan["eidx2d"]))

    lp = params["enc0"]
    hn0, fi0, fj0 = _tc_pre(h, lp["Wn"], lp["Wni"], lp["Wnj"])
    fe0 = _tc_matmul(e_g, lp["We"], zero_from=k)
    fi0_g = _sc_gather(fi0, src128)
    fj0_g = _sc_gather(fj0, dst128)
    ex0, en0 = _tc_edge(fi0_g, fj0_g, fe0, _build_s(lp["attn"], DE_H), H, e_real,
                        lp["ln_e_s"], lp["ln_e_b"])
    d0 = _sc_scatter_add(ex0, dst128, N)
    dr0 = _tc_recip_sum(d0)
    dg0 = _sc_gather(dr0, dst128)
    hn0_g = _sc_gather(hn0, src128)
    msg0 = _tc_msg(ex0, dg0, hn0_g, jnp.asarray(_build_x_mat(H, DN_H)))
    mp0 = _sc_scatter_add(msg0, dst64, N)
    h = _tc_post(mp0, h, lp["ln_n_s"], lp["ln_n_b"])
    ef = en0

    lp = params["enc1"]
    mp1, en1 = _egat(h, ef, src128, dst128, dst64, lp, H, DN_H, DE_H, e_real,
                     enc=True)
    h = _tc_post(mp1, h, lp["ln_n_s"], lp["ln_n_b"])
    ef = en1

    rep = _tc_rep(h, params["W_e2d"], mflag)
    rep_e = _tc_matmul(ef, params["W_e2d_e"])
    mpd, _ = _egat(rep, rep_e, src128, dst128, dst64, params["dec"], 1, D, DE,
                   e_real, enc=False)
    return _tc_loss(mpd, rep, x, mflag, plan["num_mask"])


# speedup vs baseline: 19.4595x; 1.1433x over previous
import functools
import math

import numpy as np
import jax
import jax.numpy as jnp
from jax import lax
from jax.experimental import pallas as pl
from jax.experimental.pallas import tpu as pltpu
from jax.experimental.pallas import tpu_sc as plsc

N = 10000
E = 320000
D = 128
DE = 16
H = 4
DN_H = 32
DE_H = 4

NC = 2
NS = 16
NW = NC * NS
CH = 128
GRAN = NW * CH

_f32 = jnp.float32
_i32 = jnp.int32


@functools.lru_cache(maxsize=None)
def _static_plan(n, num_edges):
    rng = np.random.default_rng(0)
    perm = rng.permutation(n)
    num_mask = int(0.5 * n)
    mask_nodes = perm[:num_mask]
    perm_mask = rng.permutation(num_mask)
    num_noise = int(0.15 * num_mask)
    token_nodes = mask_nodes[perm_mask[: int(0.85 * num_mask)]]
    noise_nodes = mask_nodes[perm_mask[num_mask - num_noise:]]
    noise_chosen = rng.permutation(n)[:num_noise]
    kidx = np.nonzero(rng.random(num_edges) >= 0.5)[0].astype(np.int32)
    k = len(kidx)
    e_real = k + n
    ep = math.ceil(e_real / GRAN) * GRAN

    row_src = np.arange(n, dtype=np.int32)
    row_src[noise_nodes] = noise_chosen
    tflag = np.zeros((n, 1), np.float32)
    tflag[token_nodes] = 1.0
    mflag = np.zeros((n, 1), np.float32)
    mflag[mask_nodes] = 1.0
    eidx_p = np.zeros((ep,), np.int32)
    eidx_p[:k] = kidx
    n_pad = math.ceil(n / GRAN) * GRAN
    rowsrc_p = np.zeros((n_pad,), np.int32)
    rowsrc_p[:n] = row_src
    pairs_idx = np.full((ep,), num_edges, np.int32)
    pairs_idx[:k] = kidx
    pairs_idx[k:e_real] = num_edges + np.arange(n, dtype=np.int32)
    loop_pairs = np.stack([np.arange(n, dtype=np.int32)] * 2, axis=1)
    return dict(
        k=k, e_real=e_real, ep=ep, num_mask=num_mask,
        tflag=tflag, mflag=mflag,
        eidx2d=eidx_p.reshape(-1, CH),
        rowsrc2d=rowsrc_p.reshape(-1, CH),
        pairs_idx2d=pairs_idx.reshape(-1, CH),
        loop_pairs=loop_pairs,
    )


def _sc_mesh():
    return plsc.VectorSubcoreMesh(core_axis_name="c", subcore_axis_name="s")


_SC_PARAMS = pltpu.CompilerParams(use_tc_tiling_on_sc=False)


_STAGE_BYTES = 360_000
_W = 8


def _zero_fill(zbuf, rows, r):
    zero = jnp.zeros((16,), _f32)
    per = r // 16

    def zbody(i, carry):
        zbuf[i // per, pl.ds((i % per) * 16, 16)] = zero
        return carry

    lax.fori_loop(0, rows * per, zbody, 0)


@functools.lru_cache(maxsize=None)
def _make_gather(t_rows, r, m_pad, dtype=_f32):
    rpw = m_pad // NW
    nchunks = rpw // CH
    fits = rpw * r * 4 <= _STAGE_BYTES
    nb = nchunks if fits else 4

    @functools.partial(
        pl.kernel,
        out_type=jax.ShapeDtypeStruct((m_pad, r), dtype),
        mesh=_sc_mesh(),
        compiler_params=_SC_PARAMS,
        scratch_types=[
            pltpu.VMEM((nchunks, CH), _i32),
            pltpu.VMEM((nb * CH, r), dtype),
            pltpu.SemaphoreType.DMA,
            pltpu.SemaphoreType.DMA,
            pltpu.SemaphoreType.DMA,
        ],
    )
    def gather_k(table_hbm, idx_hbm, out_hbm, idx_v, rows_v, isem, gsem, osem):
        c = lax.axis_index("c")
        s = lax.axis_index("s")
        wid = s * NC + c
        pltpu.async_copy(idx_hbm.at[pl.ds(wid * nchunks, nchunks)],
                         idx_v, isem).wait()
        if fits:
            descs = [None] * nchunks
            for j in range(nchunks):
                if j >= _W:
                    descs[j - _W].wait()
                descs[j] = pltpu.async_copy(
                    table_hbm.at[idx_v.at[j]],
                    rows_v.at[pl.ds(j * CH, CH)], gsem)
            for j in range(max(0, nchunks - _W), nchunks):
                descs[j].wait()
            pltpu.sync_copy(rows_v, out_hbm.at[pl.ds(wid * rpw, rpw)])
        else:
            lag = nb - 1
            gd = [None] * nchunks
            od = [None] * nchunks
            for i in range(nchunks + lag):
                if i < nchunks:
                    b = i % nb
                    if i >= nb:
                        od[i - nb].wait()
                    gd[i] = pltpu.async_copy(
                        table_hbm.at[idx_v.at[i]],
                        rows_v.at[pl.ds(b * CH, CH)], gsem)
                j = i - lag
                if 0 <= j < nchunks:
                    gd[j].wait()
                    od[j] = pltpu.async_copy(
                        rows_v.at[pl.ds((j % nb) * CH, CH)],
                        out_hbm.at[pl.ds(wid * rpw + j * CH, CH)], osem)
            for j in range(max(0, nchunks - nb), nchunks):
                od[j].wait()

    return gather_k


def _sc_gather(table, idx2d):
    m_pad = idx2d.shape[0] * CH
    return _make_gather(table.shape[0], table.shape[1], m_pad,
                        table.dtype)(table, idx2d)


@functools.lru_cache(maxsize=None)
def _make_scatter_add(n_out, r, m_pad):
    ch = 64 if r * 4 > 256 else CH
    rpw = m_pad // NW
    nchunks = rpw // ch
    rpt = n_out // NS
    assert rpt * NS == n_out
    zmax = 32 if r * 4 > 256 else CH
    zch = max(d for d in range(1, zmax + 1) if rpt % d == 0)
    fits = rpw * r * 4 <= _STAGE_BYTES
    nb = nchunks if fits else 4

    @functools.partial(
        pl.kernel,
        out_type=jax.ShapeDtypeStruct((NC, n_out, r), _f32),
        mesh=_sc_mesh(),
        compiler_params=_SC_PARAMS,
        scratch_types=[
            pltpu.VMEM((nchunks, ch), _i32),
            pltpu.VMEM((nb * ch, r), _f32),
            pltpu.VMEM((zch, r), _f32),
            pltpu.VMEM_SHARED((n_out, r), _f32),
            pltpu.SemaphoreType.DMA,
            pltpu.SemaphoreType.DMA,
            pltpu.SemaphoreType.DMA,
        ],
    )
    def scatter_k(vals_hbm, idx_hbm, out_hbm, idx_v, rows_v, zbuf, acc,
                  isem, vsem, ssem):
        c = lax.axis_index("c")
        s = lax.axis_index("s")
        wid = s * NC + c
        idesc = pltpu.async_copy(idx_hbm.at[pl.ds(wid * nchunks, nchunks)],
                                 idx_v, isem)
        _zero_fill(zbuf, zch, r)
        zd = [None] * (rpt // zch)
        for j in range(rpt // zch):
            zd[j] = pltpu.async_copy(
                zbuf, acc.at[pl.ds(s * rpt + j * zch, zch)], vsem)
        for d in zd:
            d.wait()
        idesc.wait()
        plsc.subcore_barrier()
        if fits:
            vdesc = pltpu.async_copy(
                vals_hbm.at[pl.ds(wid * rpw, rpw)], rows_v, vsem)
            vdesc.wait()
            sd = [None] * nchunks
            for j in range(nchunks):
                if j >= _W:
                    sd[j - _W].wait()
                sd[j] = pltpu.async_copy(
                    rows_v.at[pl.ds(j * ch, ch)],
                    acc.at[idx_v.at[j]], ssem, add=True)
            for j in range(max(0, nchunks - _W), nchunks):
                sd[j].wait()
        else:
            lag = nb - 1
            vd = [None] * nchunks
            sd = [None] * nchunks
            for i in range(nchunks + lag):
                if i < nchunks:
                    b = i % nb
                    if i >= nb:
                        sd[i - nb].wait()
                    vd[i] = pltpu.async_copy(
                        vals_hbm.at[pl.ds(wid * rpw + i * ch, ch)],
                        rows_v.at[pl.ds(b * ch, ch)], vsem)
                j = i - lag
                if 0 <= j < nchunks:
                    vd[j].wait()
                    sd[j] = pltpu.async_copy(
                        rows_v.at[pl.ds((j % nb) * ch, ch)],
                        acc.at[idx_v.at[j]], ssem, add=True)
            for j in range(max(0, nchunks - nb), nchunks):
                sd[j].wait()
        plsc.subcore_barrier()
        pltpu.sync_copy(acc.at[pl.ds(s * rpt, rpt)],
                        out_hbm.at[c].at[pl.ds(s * rpt, rpt)])

    return scatter_k


def _lane_bcast(v, lane):
    idx = jnp.full((16, 1), lane, dtype=_i32)
    dnums = lax.GatherDimensionNumbers(
        offset_dims=(), collapsed_slice_dims=(0,), start_index_map=(0,))
    return lax.gather(v, idx, dnums, (1,),
                      mode=lax.GatherScatterMode.PROMISE_IN_BOUNDS)


@functools.lru_cache(maxsize=None)
def _make_msg(n_out, m_pad, heads):
    ch = 32
    nb = 2
    rpw = m_pad // NW
    nchunks = rpw // ch
    assert nchunks % nb == 0
    rpt = n_out // NS
    zch = max(d for d in range(1, 33) if rpt % d == 0)
    vregs_per_head = D // heads // 16

    @functools.partial(
        pl.kernel,
        out_type=jax.ShapeDtypeStruct((NC, n_out, D), _f32),
        mesh=_sc_mesh(),
        compiler_params=_SC_PARAMS,
        scratch_types=[
            pltpu.VMEM((nchunks, ch), _i32),
            pltpu.VMEM((nchunks, ch), _i32),
            pltpu.VMEM((nb, ch, D), _f32),
            pltpu.VMEM((nb, ch, D), _f32),
            pltpu.VMEM((nb, ch, 16), _f32),
            pltpu.VMEM((nb, ch, 16), _f32),
            pltpu.VMEM((zch, D), _f32),
            pltpu.VMEM_SHARED((n_out, D), _f32),
            pltpu.SemaphoreType.DMA,
            pltpu.SemaphoreType.DMA,
            pltpu.SemaphoreType.DMA,
            pltpu.SemaphoreType.DMA,
            pltpu.SemaphoreType.DMA,
        ],
    )
    def msg_k(hn_hbm, ex_hbm, dr_hbm, sidx_hbm, didx_hbm, out_hbm,
              sidx, didx, hrows, mrows, erows, drrows, zbuf, acc,
              isem, hsem, esem, dsem, ssem):
        c = lax.axis_index("c")
        s = lax.axis_index("s")
        wid = s * NC + c
        i0 = pltpu.async_copy(sidx_hbm.at[pl.ds(wid * nchunks, nchunks)],
                              sidx, isem)
        i1 = pltpu.async_copy(didx_hbm.at[pl.ds(wid * nchunks, nchunks)],
                              didx, isem)
        _zero_fill(zbuf, zch, D)
        zd = [pltpu.async_copy(zbuf, acc.at[pl.ds(s * rpt + j * zch, zch)],
                               hsem)
              for j in range(rpt // zch)]
        for dsc in zd:
            dsc.wait()
        i0.wait()
        i1.wait()
        plsc.subcore_barrier()

        def fire(i, b):
            base = wid * rpw
            pltpu.async_copy(hn_hbm.at[sidx.at[i]], hrows.at[b], hsem)
            pltpu.async_copy(ex_hbm.at[pl.ds(base + i * ch, ch)],
                             erows.at[b], esem)
            pltpu.async_copy(dr_hbm.at[didx.at[i]], drrows.at[b], dsem)

        for b in range(nb):
            fire(b, b)

        def outer(g, carry):
            for b in range(nb):
                i = g * nb + b
                pltpu.make_async_copy(hn_hbm.at[pl.ds(0, ch)],
                                      hrows.at[b], hsem).wait()
                pltpu.make_async_copy(ex_hbm.at[pl.ds(0, ch)],
                                      erows.at[b], esem).wait()
                pltpu.make_async_copy(dr_hbm.at[pl.ds(0, ch)],
                                      drrows.at[b], dsem).wait()

                @pl.when(g > 0)
                def _():
                    pltpu.make_async_copy(hn_hbm.at[pl.ds(0, ch)],
                                          mrows.at[b], ssem).wait()

                def edge(e, carry2):
                    av = erows[b, e] * drrows[b, e]
                    for hh in range(heads):
                        m = _lane_bcast(av, hh)
                        for v in range(vregs_per_head):
                            off = (hh * vregs_per_head + v) * 16
                            mrows[b, e, pl.ds(off, 16)] = (
                                hrows[b, e, pl.ds(off, 16)] * m)
                    return carry2

                lax.fori_loop(0, ch, edge, 0)
                nxt = jnp.minimum(i + nb, nchunks - 1)
                fire(nxt, b)
                pltpu.async_copy(mrows.at[b], acc.at[didx.at[i]], ssem,
                                 add=True)
            return carry

        lax.fori_loop(0, nchunks // nb, outer, 0)
        for b in range(nb):
            pltpu.make_async_copy(hn_hbm.at[pl.ds(0, ch)],
                                  hrows.at[b], hsem).wait()
            pltpu.make_async_copy(ex_hbm.at[pl.ds(0, ch)],
                                  erows.at[b], esem).wait()
            pltpu.make_async_copy(dr_hbm.at[pl.ds(0, ch)],
                                  drrows.at[b], dsem).wait()
            pltpu.make_async_copy(hn_hbm.at[pl.ds(0, ch)],
                                  mrows.at[b], ssem).wait()
        plsc.subcore_barrier()
        pltpu.sync_copy(acc.at[pl.ds(s * rpt, rpt)],
                        out_hbm.at[c].at[pl.ds(s * rpt, rpt)])

    return msg_k


def _sc_msg(hn, ex16, dr16, src32, dst32, n_out, heads):
    m_pad = src32.shape[0] * 32
    return _make_msg(n_out, m_pad, heads)(hn, ex16, dr16, src32, dst32)


def _sc_scatter_add(vals, idx2d, n_out):
    m, r = vals.shape
    assert m % GRAN == 0
    ch = 64 if r * 4 > 256 else CH
    assert idx2d.shape == (m // ch, ch)
    return _make_scatter_add(n_out, r, m)(vals, idx2d)


def _full(shape):
    return pl.BlockSpec(shape, lambda i: tuple(0 for _ in shape))


def _rows(bshape):
    return pl.BlockSpec(bshape, lambda i: (i,) + tuple(0 for _ in bshape[1:]))


def _tc_mask_x(xg, tflag, mask_token):
    n = tflag.shape[0]
    b = 1000

    def body(xg_ref, tf_ref, mt_ref, out_ref):
        tf = tf_ref[...]
        out_ref[...] = xg_ref[...] * (1.0 - tf) + tf * mt_ref[...]

    return pl.pallas_call(
        body,
        grid=(n // b,),
        in_specs=[_rows((b, D)), _rows((b, 1)), _full((1, D))],
        out_specs=_rows((b, D)),
        out_shape=jax.ShapeDtypeStruct((n, D), _f32),
    )(xg, tflag, mask_token)


def _tc_pre(h, wn, wni, wnj):
    n, d = h.shape
    b = 1000
    dh = wn.shape[1]
    de16 = wni.shape[1]

    def body(h_ref, wn_ref, wni_ref, wnj_ref, hn_ref, fi_ref, fj_ref):
        hb = h_ref[...]
        hn_ref[...] = jnp.dot(hb, wn_ref[...], preferred_element_type=_f32)
        fi_ref[...] = jnp.dot(hb, wni_ref[...], preferred_element_type=_f32)
        fj_ref[...] = jnp.dot(hb, wnj_ref[...], preferred_element_type=_f32)

    return pl.pallas_call(
        body,
        grid=(n // b,),
        in_specs=[_rows((b, d)), _full((d, dh)), _full((d, de16)), _full((d, de16))],
        out_specs=[_rows((b, dh)), _rows((b, de16)), _rows((b, de16))],
        out_shape=[
            jax.ShapeDtypeStruct((n, dh), _f32),
            jax.ShapeDtypeStruct((n, de16), _f32),
            jax.ShapeDtypeStruct((n, de16), _f32),
        ],
    )(h, wn, wni, wnj)


def _tc_matmul(a, w, zero_from=None):
    m, ka = a.shape
    kb, r = w.shape
    b = 4096 if m % 4096 == 0 else 1000

    def body(a_ref, w_ref, o_ref):
        o = jnp.dot(a_ref[...], w_ref[...], preferred_element_type=_f32)
        if zero_from is not None:
            row = lax.broadcasted_iota(_i32, (b, r), 0) + pl.program_id(0) * b
            o = jnp.where(row < zero_from, o, 0.0)
        o_ref[...] = o

    return pl.pallas_call(
        body,
        grid=(m // b,),
        in_specs=[_rows((b, ka)), _full((kb, r))],
        out_specs=_rows((b, r)),
        out_shape=jax.ShapeDtypeStruct((m, r), _f32),
    )(a, w)


def _tc_edge(fi_g, fj_g, fe, s_mat, heads, e_real, ln_s=None, ln_b=None):
    m = fi_g.shape[0]
    b = 4096
    with_en = ln_s is not None

    def body(*refs):
        if with_en:
            fi_ref, fj_ref, fe_ref, s_ref, lns_ref, lnb_ref, ex_ref, en_ref = refs
        else:
            fi_ref, fj_ref, fe_ref, s_ref, ex_ref = refs
        f = fi_ref[...] + fj_ref[...] + fe_ref[...]
        f = jnp.where(f > 0, f, 0.2 * f)
        sc = jnp.dot(f, s_ref[...], preferred_element_type=_f32)
        row = lax.broadcasted_iota(_i32, (b, 16), 0) + pl.program_id(0) * b
        col = lax.broadcasted_iota(_i32, (b, 16), 1)
        keep = jnp.logical_and(row < e_real, col < heads)
        ex_ref[...] = jnp.where(keep, jnp.exp(sc), 0.0)
        if with_en:
            r0 = jnp.maximum(f, 0.0)
            mu = jnp.mean(r0, axis=-1, keepdims=True)
            var = jnp.mean(r0 * r0, axis=-1, keepdims=True) - mu * mu
            en_ref[...] = (r0 - mu) * lax.rsqrt(var + 1e-5) * lns_ref[...] + lnb_ref[...]

    in_specs = [_rows((b, 16))] * 3 + [_full((16, 16))]
    out_specs = [_rows((b, 16))]
    out_shape = [jax.ShapeDtypeStruct((m, 16), _f32)]
    args = [fi_g, fj_g, fe, s_mat]
    if with_en:
        in_specs += [_full((1, 16)), _full((1, 16))]
        args += [ln_s.reshape(1, 16), ln_b.reshape(1, 16)]
        out_specs.append(_rows((b, 16)))
        out_shape.append(jax.ShapeDtypeStruct((m, 16), _f32))
    out = pl.pallas_call(
        body,
        grid=(m // b,),
        in_specs=in_specs,
        out_specs=out_specs if with_en else out_specs[0],
        out_shape=out_shape if with_en else out_shape[0],
    )(*args)
    return out if with_en else (out, None)


def _tc_recip_sum(dparts):
    n = dparts.shape[1]
    b = 1000

    def body(d0_ref, d1_ref, o_ref):
        o_ref[...] = 1.0 / (d0_ref[...] + d1_ref[...] + 1e-9)

    return pl.pallas_call(
        body,
        grid=(n // b,),
        in_specs=[_rows((b, 16)), _rows((b, 16))],
        out_specs=_rows((b, 16)),
        out_shape=jax.ShapeDtypeStruct((n, 16), _f32),
    )(dparts[0], dparts[1])


def _tc_msg(ex16, dgr, hn_g, x_mat):
    m = hn_g.shape[0]
    b = 2048

    def body(ex_ref, dg_ref, hn_ref, x_ref, o_ref):
        a = jnp.dot(ex_ref[...] * dg_ref[...], x_ref[...],
                    preferred_element_type=_f32)
        o_ref[...] = a * hn_ref[...]

    return pl.pallas_call(
        body,
        grid=(m // b,),
        in_specs=[_rows((b, 16)), _rows((b, 16)), _rows((b, D)), _full((16, D))],
        out_specs=_rows((b, D)),
        out_shape=jax.ShapeDtypeStruct((m, D), _f32),
    )(ex16, dgr, hn_g, x_mat)


def _tc_post(mparts, h, ln_s, ln_b):
    n = h.shape[0]
    b = 1000

    def body(p0_ref, p1_ref, h_ref, s_ref, bb_ref, o_ref):
        o = p0_ref[...] + p1_ref[...] + h_ref[...]
        o = jnp.maximum(o, 0.0)
        mu = jnp.mean(o, axis=-1, keepdims=True)
        var = jnp.mean(o * o, axis=-1, keepdims=True) - mu * mu
        o_ref[...] = (o - mu) * lax.rsqrt(var + 1e-5) * s_ref[...] + bb_ref[...]

    return pl.pallas_call(
        body,
        grid=(n // b,),
        in_specs=[_rows((b, D))] * 3 + [_full((1, D)), _full((1, D))],
        out_specs=_rows((b, D)),
        out_shape=jax.ShapeDtypeStruct((n, D), _f32),
    )(mparts[0], mparts[1], h, ln_s.reshape(1, D), ln_b.reshape(1, D))


def _tc_rep(h, w, mflag):
    n = h.shape[0]
    b = 1000

    def body(h_ref, w_ref, mf_ref, o_ref):
        o = jnp.dot(h_ref[...], w_ref[...], preferred_element_type=_f32)
        o_ref[...] = o * (1.0 - mf_ref[...])

    return pl.pallas_call(
        body,
        grid=(n // b,),
        in_specs=[_rows((b, D)), _full((D, D)), _rows((b, 1))],
        out_specs=_rows((b, D)),
        out_shape=jax.ShapeDtypeStruct((n, D), _f32),
    )(h, w, mflag)


def _tc_loss(mparts, rep, x, mflag, num_mask):
    n = x.shape[0]
    b = 1000

    def body(p0_ref, p1_ref, rep_ref, x_ref, mf_ref, o_ref):
        recon = p0_ref[...] + p1_ref[...] + rep_ref[...]
        xb = x_ref[...]
        nx = jnp.sqrt(jnp.sum(recon * recon, -1, keepdims=True)) + 1e-8
        ny = jnp.sqrt(jnp.sum(xb * xb, -1, keepdims=True)) + 1e-8
        cos = jnp.sum((recon / nx) * (xb / ny), -1, keepdims=True)
        v = (1.0 - cos) ** 3 * mf_ref[...]
        psum = jnp.sum(v, axis=0, keepdims=True)

        @pl.when(pl.program_id(0) == 0)
        def _():
            o_ref[...] = jnp.zeros((1, 1), _f32)

        o_ref[...] += psum

    out = pl.pallas_call(
        body,
        grid=(n // b,),
        in_specs=[_rows((b, D))] * 4 + [_rows((b, 1))],
        out_specs=pl.BlockSpec((1, 1), lambda i: (0, 0)),
        out_shape=jax.ShapeDtypeStruct((1, 1), _f32),
    )(mparts[0], mparts[1], rep, x, mflag)
    return out[0, 0] / np.float32(num_mask)


def _build_s(attn, de):
    attn_flat = attn.reshape(-1)
    rows = np.arange(16)
    s = jnp.zeros((16, 16), _f32).at[rows, rows // de].set(attn_flat)
    return s


@functools.lru_cache(maxsize=None)
def _build_x_mat(heads, dn):
    x = np.zeros((16, heads * dn), np.float32)
    for hh in range(heads):
        x[hh, hh * dn:(hh + 1) * dn] = 1.0
    return x


def _egat(h, ef, idxs, lp, heads, de, e_real, enc=True):
    src128, dst128, src32, dst32 = idxs
    hn, fi, fj = _tc_pre(h, lp["Wn"], lp["Wni"], lp["Wnj"])
    fe = _tc_matmul(ef, lp["We"], zero_from=None)
    fi_g = _sc_gather(fi, src128)
    fj_g = _sc_gather(fj, dst128)
    s_mat = _build_s(lp["attn"], de)
    if enc:
        ex16, en = _tc_edge(fi_g, fj_g, fe, s_mat, heads, e_real,
                            lp["ln_e_s"], lp["ln_e_b"])
    else:
        ex16, en = _tc_edge(fi_g, fj_g, fe, s_mat, heads, e_real)
    dparts = _sc_scatter_add(ex16, dst128, N)
    dr16 = _tc_recip_sum(dparts)
    mparts = _sc_msg(hn, ex16, dr16, src32, dst32, N, heads)
    return mparts, en


def kernel(x, edge_index, e, params):
    n = x.shape[0]
    plan = _static_plan(n, edge_index.shape[1])
    k, e_real, ep = plan["k"], plan["e_real"], plan["ep"]

    tflag = jnp.asarray(plan["tflag"])
    mflag = jnp.asarray(plan["mflag"])

    pairs = jnp.concatenate(
        [edge_index.T, jnp.asarray(plan["loop_pairs"])], axis=0)
    table16 = jnp.pad(pairs, ((0, 0), (0, 14)))
    pairs_p = _sc_gather(table16, jnp.asarray(plan["pairs_idx2d"]))
    src_p = pairs_p[:, 0]
    dst_p = pairs_p[:, 1]
    src128 = src_p.reshape(-1, CH)
    dst128 = dst_p.reshape(-1, CH)
    src32 = src_p.reshape(-1, 32)
    dst32 = dst_p.reshape(-1, 32)
    idxs = (src128, dst128, src32, dst32)

    xg = _sc_gather(x, jnp.asarray(plan["rowsrc2d"]))
    h = _tc_mask_x(xg, tflag, params["mask_token"])

    e_g = _sc_gather(e, jnp.asarray(plan["eidx2d"]))

    lp = params["enc0"]
    hn0, fi0, fj0 = _tc_pre(h, lp["Wn"], lp["Wni"], lp["Wnj"])
    fe0 = _tc_matmul(e_g, lp["We"], zero_from=k)
    fi0_g = _sc_gather(fi0, src128)
    fj0_g = _sc_gather(fj0, dst128)
    ex0, en0 = _tc_edge(fi0_g, fj0_g, fe0, _build_s(lp["attn"], DE_H), H, e_real,
                        lp["ln_e_s"], lp["ln_e_b"])
    d0 = _sc_scatter_add(ex0, dst128, N)
    dr0 = _tc_recip_sum(d0)
    mp0 = _sc_msg(hn0, ex0, dr0, src32, dst32, N, H)
    h = _tc_post(mp0, h, lp["ln_n_s"], lp["ln_n_b"])
    ef = en0

    lp = params["enc1"]
    mp1, en1 = _egat(h, ef, idxs, lp, H, DE_H, e_real, enc=True)
    h = _tc_post(mp1, h, lp["ln_n_s"], lp["ln_n_b"])
    ef = en1

    rep = _tc_rep(h, params["W_e2d"], mflag)
    rep_e = _tc_matmul(ef, params["W_e2d_e"])
    mpd, _ = _egat(rep, rep_e, idxs, params["dec"], 1, DE, e_real, enc=False)
    return _tc_loss(mpd, rep, x, mflag, plan["num_mask"])


# speedup vs baseline: 19.5409x; 1.0042x over previous
import functools
import math

import numpy as np
import jax
import jax.numpy as jnp
from jax import lax
from jax.experimental import pallas as pl
from jax.experimental.pallas import tpu as pltpu
from jax.experimental.pallas import tpu_sc as plsc

N = 10000
E = 320000
D = 128
DE = 16
H = 4
DN_H = 32
DE_H = 4

NC = 2
NS = 16
NW = NC * NS
CH = 128
GRAN = NW * CH

_f32 = jnp.float32
_i32 = jnp.int32


@functools.lru_cache(maxsize=None)
def _static_plan(n, num_edges):
    rng = np.random.default_rng(0)
    perm = rng.permutation(n)
    num_mask = int(0.5 * n)
    mask_nodes = perm[:num_mask]
    perm_mask = rng.permutation(num_mask)
    num_noise = int(0.15 * num_mask)
    token_nodes = mask_nodes[perm_mask[: int(0.85 * num_mask)]]
    noise_nodes = mask_nodes[perm_mask[num_mask - num_noise:]]
    noise_chosen = rng.permutation(n)[:num_noise]
    kidx = np.nonzero(rng.random(num_edges) >= 0.5)[0].astype(np.int32)
    k = len(kidx)
    e_real = k + n
    ep = math.ceil(e_real / GRAN) * GRAN

    row_src = np.arange(n, dtype=np.int32)
    row_src[noise_nodes] = noise_chosen
    tflag = np.zeros((n, 1), np.float32)
    tflag[token_nodes] = 1.0
    mflag = np.zeros((n, 1), np.float32)
    mflag[mask_nodes] = 1.0
    eidx_p = np.zeros((ep,), np.int32)
    eidx_p[:k] = kidx
    n_pad = math.ceil(n / GRAN) * GRAN
    rowsrc_p = np.zeros((n_pad,), np.int32)
    rowsrc_p[:n] = row_src
    pairs_idx = np.full((ep,), num_edges, np.int32)
    pairs_idx[:k] = kidx
    pairs_idx[k:e_real] = num_edges + np.arange(n, dtype=np.int32)
    loop_pairs = np.stack([np.arange(n, dtype=np.int32)] * 2, axis=1)
    return dict(
        k=k, e_real=e_real, ep=ep, num_mask=num_mask,
        tflag=tflag, mflag=mflag,
        eidx2d=eidx_p.reshape(-1, CH),
        rowsrc2d=rowsrc_p.reshape(-1, CH),
        pairs_idx2d=pairs_idx.reshape(-1, CH),
        loop_pairs=loop_pairs,
    )


def _sc_mesh():
    return plsc.VectorSubcoreMesh(core_axis_name="c", subcore_axis_name="s")


_SC_PARAMS = pltpu.CompilerParams(use_tc_tiling_on_sc=False)


_STAGE_BYTES = 360_000
_W = 8


def _zero_fill(zbuf, rows, r):
    zero = jnp.zeros((16,), _f32)
    per = r // 16

    def zbody(i, carry):
        zbuf[i // per, pl.ds((i % per) * 16, 16)] = zero
        return carry

    lax.fori_loop(0, rows * per, zbody, 0)


@functools.lru_cache(maxsize=None)
def _make_gather(t_rows, r, m_pad, dtype=_f32):
    rpw = m_pad // NW
    nchunks = rpw // CH
    fits = rpw * r * 4 <= _STAGE_BYTES
    nb = nchunks if fits else 4

    @functools.partial(
        pl.kernel,
        out_type=jax.ShapeDtypeStruct((m_pad, r), dtype),
        mesh=_sc_mesh(),
        compiler_params=_SC_PARAMS,
        scratch_types=[
            pltpu.VMEM((nchunks, CH), _i32),
            pltpu.VMEM((nb * CH, r), dtype),
            pltpu.SemaphoreType.DMA,
            pltpu.SemaphoreType.DMA,
            pltpu.SemaphoreType.DMA,
        ],
    )
    def gather_k(table_hbm, idx_hbm, out_hbm, idx_v, rows_v, isem, gsem, osem):
        c = lax.axis_index("c")
        s = lax.axis_index("s")
        wid = s * NC + c
        pltpu.async_copy(idx_hbm.at[pl.ds(wid * nchunks, nchunks)],
                         idx_v, isem).wait()
        if fits:
            descs = [None] * nchunks
            for j in range(nchunks):
                if j >= _W:
                    descs[j - _W].wait()
                descs[j] = pltpu.async_copy(
                    table_hbm.at[idx_v.at[j]],
                    rows_v.at[pl.ds(j * CH, CH)], gsem)
            for j in range(max(0, nchunks - _W), nchunks):
                descs[j].wait()
            pltpu.sync_copy(rows_v, out_hbm.at[pl.ds(wid * rpw, rpw)])
        else:
            lag = nb - 1
            gd = [None] * nchunks
            od = [None] * nchunks
            for i in range(nchunks + lag):
                if i < nchunks:
                    b = i % nb
                    if i >= nb:
                        od[i - nb].wait()
                    gd[i] = pltpu.async_copy(
                        table_hbm.at[idx_v.at[i]],
                        rows_v.at[pl.ds(b * CH, CH)], gsem)
                j = i - lag
                if 0 <= j < nchunks:
                    gd[j].wait()
                    od[j] = pltpu.async_copy(
                        rows_v.at[pl.ds((j % nb) * CH, CH)],
                        out_hbm.at[pl.ds(wid * rpw + j * CH, CH)], osem)
            for j in range(max(0, nchunks - nb), nchunks):
                od[j].wait()

    return gather_k


def _sc_gather(table, idx2d):
    m_pad = idx2d.shape[0] * CH
    return _make_gather(table.shape[0], table.shape[1], m_pad,
                        table.dtype)(table, idx2d)


@functools.lru_cache(maxsize=None)
def _make_scatter_add(n_out, r, m_pad):
    ch = 64 if r * 4 > 256 else CH
    rpw = m_pad // NW
    nchunks = rpw // ch
    rpt = n_out // NS
    assert rpt * NS == n_out
    zmax = 32 if r * 4 > 256 else CH
    zch = max(d for d in range(1, zmax + 1) if rpt % d == 0)
    fits = rpw * r * 4 <= _STAGE_BYTES
    nb = nchunks if fits else 4

    @functools.partial(
        pl.kernel,
        out_type=jax.ShapeDtypeStruct((NC, n_out, r), _f32),
        mesh=_sc_mesh(),
        compiler_params=_SC_PARAMS,
        scratch_types=[
            pltpu.VMEM((nchunks, ch), _i32),
            pltpu.VMEM((nb * ch, r), _f32),
            pltpu.VMEM((zch, r), _f32),
            pltpu.VMEM_SHARED((n_out, r), _f32),
            pltpu.SemaphoreType.DMA,
            pltpu.SemaphoreType.DMA,
            pltpu.SemaphoreType.DMA,
        ],
    )
    def scatter_k(vals_hbm, idx_hbm, out_hbm, idx_v, rows_v, zbuf, acc,
                  isem, vsem, ssem):
        c = lax.axis_index("c")
        s = lax.axis_index("s")
        wid = s * NC + c
        idesc = pltpu.async_copy(idx_hbm.at[pl.ds(wid * nchunks, nchunks)],
                                 idx_v, isem)
        _zero_fill(zbuf, zch, r)
        zd = [None] * (rpt // zch)
        for j in range(rpt // zch):
            zd[j] = pltpu.async_copy(
                zbuf, acc.at[pl.ds(s * rpt + j * zch, zch)], vsem)
        for d in zd:
            d.wait()
        idesc.wait()
        plsc.subcore_barrier()
        if fits:
            vdesc = pltpu.async_copy(
                vals_hbm.at[pl.ds(wid * rpw, rpw)], rows_v, vsem)
            vdesc.wait()
            sd = [None] * nchunks
            for j in range(nchunks):
                if j >= _W:
                    sd[j - _W].wait()
                sd[j] = pltpu.async_copy(
                    rows_v.at[pl.ds(j * ch, ch)],
                    acc.at[idx_v.at[j]], ssem, add=True)
            for j in range(max(0, nchunks - _W), nchunks):
                sd[j].wait()
        else:
            lag = nb - 1
            vd = [None] * nchunks
            sd = [None] * nchunks
            for i in range(nchunks + lag):
                if i < nchunks:
                    b = i % nb
                    if i >= nb:
                        sd[i - nb].wait()
                    vd[i] = pltpu.async_copy(
                        vals_hbm.at[pl.ds(wid * rpw + i * ch, ch)],
                        rows_v.at[pl.ds(b * ch, ch)], vsem)
                j = i - lag
                if 0 <= j < nchunks:
                    vd[j].wait()
                    sd[j] = pltpu.async_copy(
                        rows_v.at[pl.ds((j % nb) * ch, ch)],
                        acc.at[idx_v.at[j]], ssem, add=True)
            for j in range(max(0, nchunks - nb), nchunks):
                sd[j].wait()
        plsc.subcore_barrier()
        pltpu.sync_copy(acc.at[pl.ds(s * rpt, rpt)],
                        out_hbm.at[c].at[pl.ds(s * rpt, rpt)])

    return scatter_k


def _lane_bcast(v, lane):
    idx = jnp.full((16, 1), lane, dtype=_i32)
    dnums = lax.GatherDimensionNumbers(
        offset_dims=(), collapsed_slice_dims=(0,), start_index_map=(0,))
    return lax.gather(v, idx, dnums, (1,),
                      mode=lax.GatherScatterMode.PROMISE_IN_BOUNDS)


@functools.lru_cache(maxsize=None)
def _make_msg(n_out, m_pad, heads):
    ch = 48
    nb = 2
    rpw = m_pad // NW
    nchunks = rpw // ch
    assert nchunks % nb == 0
    rpt = n_out // NS
    zch = max(d for d in range(1, 33) if rpt % d == 0)
    vregs_per_head = D // heads // 16

    @functools.partial(
        pl.kernel,
        out_type=jax.ShapeDtypeStruct((NC, n_out, D), _f32),
        mesh=_sc_mesh(),
        compiler_params=_SC_PARAMS,
        scratch_types=[
            pltpu.VMEM((nchunks, ch), _i32),
            pltpu.VMEM((nchunks, ch), _i32),
            pltpu.VMEM((nb, ch, D), _f32),
            pltpu.VMEM((nb, ch, D), _f32),
            pltpu.VMEM((nb, ch, 16), _f32),
            pltpu.VMEM((nb, ch, 16), _f32),
            pltpu.VMEM((zch, D), _f32),
            pltpu.VMEM_SHARED((n_out, D), _f32),
            pltpu.SemaphoreType.DMA,
            pltpu.SemaphoreType.DMA,
            pltpu.SemaphoreType.DMA,
            pltpu.SemaphoreType.DMA,
            pltpu.SemaphoreType.DMA,
        ],
    )
    def msg_k(hn_hbm, ex_hbm, dr_hbm, sidx_hbm, didx_hbm, out_hbm,
              sidx, didx, hrows, mrows, erows, drrows, zbuf, acc,
              isem, hsem, esem, dsem, ssem):
        c = lax.axis_index("c")
        s = lax.axis_index("s")
        wid = s * NC + c
        i0 = pltpu.async_copy(sidx_hbm.at[pl.ds(wid * nchunks, nchunks)],
                              sidx, isem)
        i1 = pltpu.async_copy(didx_hbm.at[pl.ds(wid * nchunks, nchunks)],
                              didx, isem)
        _zero_fill(zbuf, zch, D)
        zd = [pltpu.async_copy(zbuf, acc.at[pl.ds(s * rpt + j * zch, zch)],
                               hsem)
              for j in range(rpt // zch)]
        for dsc in zd:
            dsc.wait()
        i0.wait()
        i1.wait()
        plsc.subcore_barrier()

        def fire(i, b):
            base = wid * rpw
            pltpu.async_copy(hn_hbm.at[sidx.at[i]], hrows.at[b], hsem)
            pltpu.async_copy(ex_hbm.at[pl.ds(base + i * ch, ch)],
                             erows.at[b], esem)
            pltpu.async_copy(dr_hbm.at[didx.at[i]], drrows.at[b], dsem)

        for b in range(nb):
            fire(b, b)

        def outer(g, carry):
            for b in range(nb):
                i = g * nb + b
                pltpu.make_async_copy(hn_hbm.at[pl.ds(0, ch)],
                                      hrows.at[b], hsem).wait()
                pltpu.make_async_copy(ex_hbm.at[pl.ds(0, ch)],
                                      erows.at[b], esem).wait()
                pltpu.make_async_copy(dr_hbm.at[pl.ds(0, ch)],
                                      drrows.at[b], dsem).wait()

                @pl.when(g > 0)
                def _():
                    pltpu.make_async_copy(hn_hbm.at[pl.ds(0, ch)],
                                          mrows.at[b], ssem).wait()

                def edge(e, carry2):
                    av = erows[b, e] * drrows[b, e]
                    for hh in range(heads):
                        m = _lane_bcast(av, hh)
                        for v in range(vregs_per_head):
                            off = (hh * vregs_per_head + v) * 16
                            mrows[b, e, pl.ds(off, 16)] = (
                                hrows[b, e, pl.ds(off, 16)] * m)
                    return carry2

                lax.fori_loop(0, ch, edge, 0, unroll=4)
                nxt = jnp.minimum(i + nb, nchunks - 1)
                fire(nxt, b)
                pltpu.async_copy(mrows.at[b], acc.at[didx.at[i]], ssem,
                                 add=True)
            return carry

        lax.fori_loop(0, nchunks // nb, outer, 0)
        for b in range(nb):
            pltpu.make_async_copy(hn_hbm.at[pl.ds(0, ch)],
                                  hrows.at[b], hsem).wait()
            pltpu.make_async_copy(ex_hbm.at[pl.ds(0, ch)],
                                  erows.at[b], esem).wait()
            pltpu.make_async_copy(dr_hbm.at[pl.ds(0, ch)],
                                  drrows.at[b], dsem).wait()
            pltpu.make_async_copy(hn_hbm.at[pl.ds(0, ch)],
                                  mrows.at[b], ssem).wait()
        plsc.subcore_barrier()
        pltpu.sync_copy(acc.at[pl.ds(s * rpt, rpt)],
                        out_hbm.at[c].at[pl.ds(s * rpt, rpt)])

    return msg_k


def _sc_msg(hn, ex16, dr16, src48, dst48, n_out, heads):
    m_pad = src48.shape[0] * 48
    return _make_msg(n_out, m_pad, heads)(hn, ex16, dr16, src48, dst48)


def _sc_scatter_add(vals, idx2d, n_out):
    m, r = vals.shape
    assert m % GRAN == 0
    ch = 64 if r * 4 > 256 else CH
    assert idx2d.shape == (m // ch, ch)
    return _make_scatter_add(n_out, r, m)(vals, idx2d)


def _full(shape):
    return pl.BlockSpec(shape, lambda i: tuple(0 for _ in shape))


def _rows(bshape):
    return pl.BlockSpec(bshape, lambda i: (i,) + tuple(0 for _ in bshape[1:]))


def _tc_mask_x(xg, tflag, mask_token):
    n = tflag.shape[0]
    b = 1000

    def body(xg_ref, tf_ref, mt_ref, out_ref):
        tf = tf_ref[...]
        out_ref[...] = xg_ref[...] * (1.0 - tf) + tf * mt_ref[...]

    return pl.pallas_call(
        body,
        grid=(n // b,),
        in_specs=[_rows((b, D)), _rows((b, 1)), _full((1, D))],
        out_specs=_rows((b, D)),
        out_shape=jax.ShapeDtypeStruct((n, D), _f32),
    )(xg, tflag, mask_token)


def _tc_pre(h, wn, wni, wnj):
    n, d = h.shape
    b = 1000
    dh = wn.shape[1]
    de16 = wni.shape[1]

    def body(h_ref, wn_ref, wni_ref, wnj_ref, hn_ref, fi_ref, fj_ref):
        hb = h_ref[...]
        hn_ref[...] = jnp.dot(hb, wn_ref[...], preferred_element_type=_f32)
        fi_ref[...] = jnp.dot(hb, wni_ref[...], preferred_element_type=_f32)
        fj_ref[...] = jnp.dot(hb, wnj_ref[...], preferred_element_type=_f32)

    return pl.pallas_call(
        body,
        grid=(n // b,),
        in_specs=[_rows((b, d)), _full((d, dh)), _full((d, de16)), _full((d, de16))],
        out_specs=[_rows((b, dh)), _rows((b, de16)), _rows((b, de16))],
        out_shape=[
            jax.ShapeDtypeStruct((n, dh), _f32),
            jax.ShapeDtypeStruct((n, de16), _f32),
            jax.ShapeDtypeStruct((n, de16), _f32),
        ],
    )(h, wn, wni, wnj)


def _tc_matmul(a, w, zero_from=None):
    m, ka = a.shape
    kb, r = w.shape
    b = 4096 if m % 4096 == 0 else 1000

    def body(a_ref, w_ref, o_ref):
        o = jnp.dot(a_ref[...], w_ref[...], preferred_element_type=_f32)
        if zero_from is not None:
            row = lax.broadcasted_iota(_i32, (b, r), 0) + pl.program_id(0) * b
            o = jnp.where(row < zero_from, o, 0.0)
        o_ref[...] = o

    return pl.pallas_call(
        body,
        grid=(m // b,),
        in_specs=[_rows((b, ka)), _full((kb, r))],
        out_specs=_rows((b, r)),
        out_shape=jax.ShapeDtypeStruct((m, r), _f32),
    )(a, w)


def _tc_edge(fi_g, fj_g, fe, s_mat, heads, e_real, ln_s=None, ln_b=None):
    m = fi_g.shape[0]
    b = 4096
    with_en = ln_s is not None

    def body(*refs):
        if with_en:
            fi_ref, fj_ref, fe_ref, s_ref, lns_ref, lnb_ref, ex_ref, en_ref = refs
        else:
            fi_ref, fj_ref, fe_ref, s_ref, ex_ref = refs
        f = fi_ref[...] + fj_ref[...] + fe_ref[...]
        f = jnp.where(f > 0, f, 0.2 * f)
        sc = jnp.dot(f, s_ref[...], preferred_element_type=_f32)
        row = lax.broadcasted_iota(_i32, (b, 16), 0) + pl.program_id(0) * b
        col = lax.broadcasted_iota(_i32, (b, 16), 1)
        keep = jnp.logical_and(row < e_real, col < heads)
        ex_ref[...] = jnp.where(keep, jnp.exp(sc), 0.0)
        if with_en:
            r0 = jnp.maximum(f, 0.0)
            mu = jnp.mean(r0, axis=-1, keepdims=True)
            var = jnp.mean(r0 * r0, axis=-1, keepdims=True) - mu * mu
            en_ref[...] = (r0 - mu) * lax.rsqrt(var + 1e-5) * lns_ref[...] + lnb_ref[...]

    in_specs = [_rows((b, 16))] * 3 + [_full((16, 16))]
    out_specs = [_rows((b, 16))]
    out_shape = [jax.ShapeDtypeStruct((m, 16), _f32)]
    args = [fi_g, fj_g, fe, s_mat]
    if with_en:
        in_specs += [_full((1, 16)), _full((1, 16))]
        args += [ln_s.reshape(1, 16), ln_b.reshape(1, 16)]
        out_specs.append(_rows((b, 16)))
        out_shape.append(jax.ShapeDtypeStruct((m, 16), _f32))
    out = pl.pallas_call(
        body,
        grid=(m // b,),
        in_specs=in_specs,
        out_specs=out_specs if with_en else out_specs[0],
        out_shape=out_shape if with_en else out_shape[0],
    )(*args)
    return out if with_en else (out, None)


def _tc_recip_sum(dparts):
    n = dparts.shape[1]
    b = 1000

    def body(d0_ref, d1_ref, o_ref):
        o_ref[...] = 1.0 / (d0_ref[...] + d1_ref[...] + 1e-9)

    return pl.pallas_call(
        body,
        grid=(n // b,),
        in_specs=[_rows((b, 16)), _rows((b, 16))],
        out_specs=_rows((b, 16)),
        out_shape=jax.ShapeDtypeStruct((n, 16), _f32),
    )(dparts[0], dparts[1])


def _tc_msg(ex16, dgr, hn_g, x_mat):
    m = hn_g.shape[0]
    b = 2048

    def body(ex_ref, dg_ref, hn_ref, x_ref, o_ref):
        a = jnp.dot(ex_ref[...] * dg_ref[...], x_ref[...],
                    preferred_element_type=_f32)
        o_ref[...] = a * hn_ref[...]

    return pl.pallas_call(
        body,
        grid=(m // b,),
        in_specs=[_rows((b, 16)), _rows((b, 16)), _rows((b, D)), _full((16, D))],
        out_specs=_rows((b, D)),
        out_shape=jax.ShapeDtypeStruct((m, D), _f32),
    )(ex16, dgr, hn_g, x_mat)


def _tc_post(mparts, h, ln_s, ln_b):
    n = h.shape[0]
    b = 1000

    def body(p0_ref, p1_ref, h_ref, s_ref, bb_ref, o_ref):
        o = p0_ref[...] + p1_ref[...] + h_ref[...]
        o = jnp.maximum(o, 0.0)
        mu = jnp.mean(o, axis=-1, keepdims=True)
        var = jnp.mean(o * o, axis=-1, keepdims=True) - mu * mu
        o_ref[...] = (o - mu) * lax.rsqrt(var + 1e-5) * s_ref[...] + bb_ref[...]

    return pl.pallas_call(
        body,
        grid=(n // b,),
        in_specs=[_rows((b, D))] * 3 + [_full((1, D)), _full((1, D))],
        out_specs=_rows((b, D)),
        out_shape=jax.ShapeDtypeStruct((n, D), _f32),
    )(mparts[0], mparts[1], h, ln_s.reshape(1, D), ln_b.reshape(1, D))


def _tc_rep(h, w, mflag):
    n = h.shape[0]
    b = 1000

    def body(h_ref, w_ref, mf_ref, o_ref):
        o = jnp.dot(h_ref[...], w_ref[...], preferred_element_type=_f32)
        o_ref[...] = o * (1.0 - mf_ref[...])

    return pl.pallas_call(
        body,
        grid=(n // b,),
        in_specs=[_rows((b, D)), _full((D, D)), _rows((b, 1))],
        out_specs=_rows((b, D)),
        out_shape=jax.ShapeDtypeStruct((n, D), _f32),
    )(h, w, mflag)


def _tc_loss(mparts, rep, x, mflag, num_mask):
    n = x.shape[0]
    b = 1000

    def body(p0_ref, p1_ref, rep_ref, x_ref, mf_ref, o_ref):
        recon = p0_ref[...] + p1_ref[...] + rep_ref[...]
        xb = x_ref[...]
        nx = jnp.sqrt(jnp.sum(recon * recon, -1, keepdims=True)) + 1e-8
        ny = jnp.sqrt(jnp.sum(xb * xb, -1, keepdims=True)) + 1e-8
        cos = jnp.sum((recon / nx) * (xb / ny), -1, keepdims=True)
        v = (1.0 - cos) ** 3 * mf_ref[...]
        psum = jnp.sum(v, axis=0, keepdims=True)

        @pl.when(pl.program_id(0) == 0)
        def _():
            o_ref[...] = jnp.zeros((1, 1), _f32)

        o_ref[...] += psum

    out = pl.pallas_call(
        body,
        grid=(n // b,),
        in_specs=[_rows((b, D))] * 4 + [_rows((b, 1))],
        out_specs=pl.BlockSpec((1, 1), lambda i: (0, 0)),
        out_shape=jax.ShapeDtypeStruct((1, 1), _f32),
    )(mparts[0], mparts[1], rep, x, mflag)
    return out[0, 0] / np.float32(num_mask)


def _build_s(attn, de):
    attn_flat = attn.reshape(-1)
    rows = np.arange(16)
    s = jnp.zeros((16, 16), _f32).at[rows, rows // de].set(attn_flat)
    return s


@functools.lru_cache(maxsize=None)
def _build_x_mat(heads, dn):
    x = np.zeros((16, heads * dn), np.float32)
    for hh in range(heads):
        x[hh, hh * dn:(hh + 1) * dn] = 1.0
    return x


def _egat(h, ef, idxs, lp, heads, de, e_real, enc=True):
    src128, dst128, src32, dst32 = idxs
    hn, fi, fj = _tc_pre(h, lp["Wn"], lp["Wni"], lp["Wnj"])
    fe = _tc_matmul(ef, lp["We"], zero_from=None)
    fi_g = _sc_gather(fi, src128)
    fj_g = _sc_gather(fj, dst128)
    s_mat = _build_s(lp["attn"], de)
    if enc:
        ex16, en = _tc_edge(fi_g, fj_g, fe, s_mat, heads, e_real,
                            lp["ln_e_s"], lp["ln_e_b"])
    else:
        ex16, en = _tc_edge(fi_g, fj_g, fe, s_mat, heads, e_real)
    dparts = _sc_scatter_add(ex16, dst128, N)
    dr16 = _tc_recip_sum(dparts)
    mparts = _sc_msg(hn, ex16, dr16, src32, dst32, N, heads)
    return mparts, en


def kernel(x, edge_index, e, params):
    n = x.shape[0]
    plan = _static_plan(n, edge_index.shape[1])
    k, e_real, ep = plan["k"], plan["e_real"], plan["ep"]

    tflag = jnp.asarray(plan["tflag"])
    mflag = jnp.asarray(plan["mflag"])

    pairs = jnp.concatenate(
        [edge_index.T, jnp.asarray(plan["loop_pairs"])], axis=0)
    table16 = jnp.pad(pairs, ((0, 0), (0, 14)))
    pairs_p = _sc_gather(table16, jnp.asarray(plan["pairs_idx2d"]))
    src_p = pairs_p[:, 0]
    dst_p = pairs_p[:, 1]
    src128 = src_p.reshape(-1, CH)
    dst128 = dst_p.reshape(-1, CH)
    src32 = src_p.reshape(-1, 48)
    dst32 = dst_p.reshape(-1, 48)
    idxs = (src128, dst128, src32, dst32)

    xg = _sc_gather(x, jnp.asarray(plan["rowsrc2d"]))
    h = _tc_mask_x(xg, tflag, params["mask_token"])

    e_g = _sc_gather(e, jnp.asarray(plan["eidx2d"]))

    lp = params["enc0"]
    hn0, fi0, fj0 = _tc_pre(h, lp["Wn"], lp["Wni"], lp["Wnj"])
    fe0 = _tc_matmul(e_g, lp["We"], zero_from=k)
    fi0_g = _sc_gather(fi0, src128)
    fj0_g = _sc_gather(fj0, dst128)
    ex0, en0 = _tc_edge(fi0_g, fj0_g, fe0, _build_s(lp["attn"], DE_H), H, e_real,
                        lp["ln_e_s"], lp["ln_e_b"])
    d0 = _sc_scatter_add(ex0, dst128, N)
    dr0 = _tc_recip_sum(d0)
    mp0 = _sc_msg(hn0, ex0, dr0, src32, dst32, N, H)
    h = _tc_post(mp0, h, lp["ln_n_s"], lp["ln_n_b"])
    ef = en0

    lp = params["enc1"]
    mp1, en1 = _egat(h, ef, idxs, lp, H, DE_H, e_real, enc=True)
    h = _tc_post(mp1, h, lp["ln_n_s"], lp["ln_n_b"])
    ef = en1

    rep = _tc_rep(h, params["W_e2d"], mflag)
    rep_e = _tc_matmul(ef, params["W_e2d_e"])
    mpd, _ = _egat(rep, rep_e, idxs, params["dec"], 1, DE, e_real, enc=False)
    return _tc_loss(mpd, rep, x, mflag, plan["num_mask"])


# speedup vs baseline: 29.0103x; 1.4846x over previous
import functools
import math

import numpy as np
import jax
import jax.numpy as jnp
from jax import lax
from jax.experimental import pallas as pl
from jax.experimental.pallas import tpu as pltpu
from jax.experimental.pallas import tpu_sc as plsc

N = 10000
E = 320000
D = 128
DE = 16
H = 4
DN_H = 32
DE_H = 4

NC = 2
NS = 16
NW = NC * NS
CH = 128
GRAN = NW * CH

_f32 = jnp.float32
_i32 = jnp.int32


@functools.lru_cache(maxsize=None)
def _static_plan(n, num_edges):
    rng = np.random.default_rng(0)
    perm = rng.permutation(n)
    num_mask = int(0.5 * n)
    mask_nodes = perm[:num_mask]
    perm_mask = rng.permutation(num_mask)
    num_noise = int(0.15 * num_mask)
    token_nodes = mask_nodes[perm_mask[: int(0.85 * num_mask)]]
    noise_nodes = mask_nodes[perm_mask[num_mask - num_noise:]]
    noise_chosen = rng.permutation(n)[:num_noise]
    kidx = np.nonzero(rng.random(num_edges) >= 0.5)[0].astype(np.int32)
    k = len(kidx)
    e_real = k + n
    ep = math.ceil(e_real / GRAN) * GRAN

    row_src = np.arange(n, dtype=np.int32)
    row_src[noise_nodes] = noise_chosen
    tflag = np.zeros((n, 1), np.float32)
    tflag[token_nodes] = 1.0
    mflag = np.zeros((n, 1), np.float32)
    mflag[mask_nodes] = 1.0
    eidx_p = np.zeros((ep,), np.int32)
    eidx_p[:k] = kidx
    n_pad = math.ceil(n / GRAN) * GRAN
    rowsrc_p = np.zeros((n_pad,), np.int32)
    rowsrc_p[:n] = row_src
    pairs_idx = np.full((ep,), num_edges, np.int32)
    pairs_idx[:k] = kidx
    pairs_idx[k:e_real] = num_edges + np.arange(n, dtype=np.int32)
    loop_pairs = np.stack([np.arange(n, dtype=np.int32)] * 2, axis=1)
    return dict(
        k=k, e_real=e_real, ep=ep, num_mask=num_mask,
        tflag=tflag, mflag=mflag,
        eidx2d=eidx_p.reshape(-1, CH),
        rowsrc2d=rowsrc_p.reshape(-1, CH),
        pairs_idx2d=pairs_idx.reshape(-1, CH),
        loop_pairs=loop_pairs,
    )


def _sc_mesh():
    return plsc.VectorSubcoreMesh(core_axis_name="c", subcore_axis_name="s")


_SC_PARAMS = pltpu.CompilerParams(use_tc_tiling_on_sc=False)


_STAGE_BYTES = 360_000
_W = 8


def _zero_fill(zbuf, rows, r):
    zero = jnp.zeros((16,), _f32)
    per = r // 16

    def zbody(i, carry):
        zbuf[i // per, pl.ds((i % per) * 16, 16)] = zero
        return carry

    lax.fori_loop(0, rows * per, zbody, 0)


@functools.lru_cache(maxsize=None)
def _make_gather(t_rows, r, m_pad, dtype=_f32):
    rpw = m_pad // NW
    nchunks = rpw // CH
    fits = rpw * r * 4 <= _STAGE_BYTES
    nb = nchunks if fits else 4

    @functools.partial(
        pl.kernel,
        out_type=jax.ShapeDtypeStruct((m_pad, r), dtype),
        mesh=_sc_mesh(),
        compiler_params=_SC_PARAMS,
        scratch_types=[
            pltpu.VMEM((nchunks, CH), _i32),
            pltpu.VMEM((nb * CH, r), dtype),
            pltpu.SemaphoreType.DMA,
            pltpu.SemaphoreType.DMA,
            pltpu.SemaphoreType.DMA,
        ],
    )
    def gather_k(table_hbm, idx_hbm, out_hbm, idx_v, rows_v, isem, gsem, osem):
        c = lax.axis_index("c")
        s = lax.axis_index("s")
        wid = s * NC + c
        pltpu.async_copy(idx_hbm.at[pl.ds(wid * nchunks, nchunks)],
                         idx_v, isem).wait()
        if fits:
            descs = [None] * nchunks
            for j in range(nchunks):
                if j >= _W:
                    descs[j - _W].wait()
                descs[j] = pltpu.async_copy(
                    table_hbm.at[idx_v.at[j]],
                    rows_v.at[pl.ds(j * CH, CH)], gsem)
            for j in range(max(0, nchunks - _W), nchunks):
                descs[j].wait()
            pltpu.sync_copy(rows_v, out_hbm.at[pl.ds(wid * rpw, rpw)])
        else:
            lag = nb - 1
            gd = [None] * nchunks
            od = [None] * nchunks
            for i in range(nchunks + lag):
                if i < nchunks:
                    b = i % nb
                    if i >= nb:
                        od[i - nb].wait()
                    gd[i] = pltpu.async_copy(
                        table_hbm.at[idx_v.at[i]],
                        rows_v.at[pl.ds(b * CH, CH)], gsem)
                j = i - lag
                if 0 <= j < nchunks:
                    gd[j].wait()
                    od[j] = pltpu.async_copy(
                        rows_v.at[pl.ds((j % nb) * CH, CH)],
                        out_hbm.at[pl.ds(wid * rpw + j * CH, CH)], osem)
            for j in range(max(0, nchunks - nb), nchunks):
                od[j].wait()

    return gather_k


def _sc_gather(table, idx2d):
    m_pad = idx2d.shape[0] * CH
    return _make_gather(table.shape[0], table.shape[1], m_pad,
                        table.dtype)(table, idx2d)


@functools.lru_cache(maxsize=None)
def _make_scatter_add(n_out, r, m_pad):
    ch = 64 if r * 4 > 256 else CH
    rpw = m_pad // NW
    nchunks = rpw // ch
    rpt = n_out // NS
    assert rpt * NS == n_out
    zmax = 32 if r * 4 > 256 else CH
    zch = max(d for d in range(1, zmax + 1) if rpt % d == 0)
    fits = rpw * r * 4 <= _STAGE_BYTES
    nb = nchunks if fits else 4

    @functools.partial(
        pl.kernel,
        out_type=jax.ShapeDtypeStruct((NC, n_out, r), _f32),
        mesh=_sc_mesh(),
        compiler_params=_SC_PARAMS,
        scratch_types=[
            pltpu.VMEM((nchunks, ch), _i32),
            pltpu.VMEM((nb * ch, r), _f32),
            pltpu.VMEM((zch, r), _f32),
            pltpu.VMEM_SHARED((n_out, r), _f32),
            pltpu.SemaphoreType.DMA,
            pltpu.SemaphoreType.DMA,
            pltpu.SemaphoreType.DMA,
        ],
    )
    def scatter_k(vals_hbm, idx_hbm, out_hbm, idx_v, rows_v, zbuf, acc,
                  isem, vsem, ssem):
        c = lax.axis_index("c")
        s = lax.axis_index("s")
        wid = s * NC + c
        idesc = pltpu.async_copy(idx_hbm.at[pl.ds(wid * nchunks, nchunks)],
                                 idx_v, isem)
        _zero_fill(zbuf, zch, r)
        zd = [None] * (rpt // zch)
        for j in range(rpt // zch):
            zd[j] = pltpu.async_copy(
                zbuf, acc.at[pl.ds(s * rpt + j * zch, zch)], vsem)
        for d in zd:
            d.wait()
        idesc.wait()
        plsc.subcore_barrier()
        if fits:
            vdesc = pltpu.async_copy(
                vals_hbm.at[pl.ds(wid * rpw, rpw)], rows_v, vsem)
            vdesc.wait()
            sd = [None] * nchunks
            for j in range(nchunks):
                if j >= _W:
                    sd[j - _W].wait()
                sd[j] = pltpu.async_copy(
                    rows_v.at[pl.ds(j * ch, ch)],
                    acc.at[idx_v.at[j]], ssem, add=True)
            for j in range(max(0, nchunks - _W), nchunks):
                sd[j].wait()
        else:
            lag = nb - 1
            vd = [None] * nchunks
            sd = [None] * nchunks
            for i in range(nchunks + lag):
                if i < nchunks:
                    b = i % nb
                    if i >= nb:
                        sd[i - nb].wait()
                    vd[i] = pltpu.async_copy(
                        vals_hbm.at[pl.ds(wid * rpw + i * ch, ch)],
                        rows_v.at[pl.ds(b * ch, ch)], vsem)
                j = i - lag
                if 0 <= j < nchunks:
                    vd[j].wait()
                    sd[j] = pltpu.async_copy(
                        rows_v.at[pl.ds((j % nb) * ch, ch)],
                        acc.at[idx_v.at[j]], ssem, add=True)
            for j in range(max(0, nchunks - nb), nchunks):
                sd[j].wait()
        plsc.subcore_barrier()
        pltpu.sync_copy(acc.at[pl.ds(s * rpt, rpt)],
                        out_hbm.at[c].at[pl.ds(s * rpt, rpt)])

    return scatter_k


def _lane_bcast(v, lane):
    idx = jnp.full((16, 1), lane, dtype=_i32)
    dnums = lax.GatherDimensionNumbers(
        offset_dims=(), collapsed_slice_dims=(0,), start_index_map=(0,))
    return lax.gather(v, idx, dnums, (1,),
                      mode=lax.GatherScatterMode.PROMISE_IN_BOUNDS)


@functools.lru_cache(maxsize=None)
def _make_msg(n_out, m_pad, heads):
    ch = 48
    nb = 2
    rpw = m_pad // NW
    nchunks = rpw // ch
    assert nchunks % nb == 0
    rpt = n_out // NS
    zch = max(d for d in range(1, 33) if rpt % d == 0)
    vregs_per_head = D // heads // 16

    @functools.partial(
        pl.kernel,
        out_type=jax.ShapeDtypeStruct((NC, n_out, D), _f32),
        mesh=_sc_mesh(),
        compiler_params=_SC_PARAMS,
        scratch_types=[
            pltpu.VMEM((nchunks, ch), _i32),
            pltpu.VMEM((nchunks, ch), _i32),
            pltpu.VMEM((nb, ch, D), _f32),
            pltpu.VMEM((nb, ch, D), _f32),
            pltpu.VMEM((nb, ch, 16), _f32),
            pltpu.VMEM((nb, ch, 16), _f32),
            pltpu.VMEM((zch, D), _f32),
            pltpu.VMEM_SHARED((n_out, D), _f32),
            pltpu.SemaphoreType.DMA,
            pltpu.SemaphoreType.DMA,
            pltpu.SemaphoreType.DMA,
            pltpu.SemaphoreType.DMA,
            pltpu.SemaphoreType.DMA,
        ],
    )
    def msg_k(hn_hbm, ex_hbm, dr_hbm, sidx_hbm, didx_hbm, out_hbm,
              sidx, didx, hrows, mrows, erows, drrows, zbuf, acc,
              isem, hsem, esem, dsem, ssem):
        c = lax.axis_index("c")
        s = lax.axis_index("s")
        wid = s * NC + c
        i0 = pltpu.async_copy(sidx_hbm.at[pl.ds(wid * nchunks, nchunks)],
                              sidx, isem)
        i1 = pltpu.async_copy(didx_hbm.at[pl.ds(wid * nchunks, nchunks)],
                              didx, isem)
        _zero_fill(zbuf, zch, D)
        zd = [pltpu.async_copy(zbuf, acc.at[pl.ds(s * rpt + j * zch, zch)],
                               hsem)
              for j in range(rpt // zch)]
        for dsc in zd:
            dsc.wait()
        i0.wait()
        i1.wait()
        plsc.subcore_barrier()

        def fire(i, b):
            base = wid * rpw
            pltpu.async_copy(hn_hbm.at[sidx.at[i]], hrows.at[b], hsem)
            pltpu.async_copy(ex_hbm.at[pl.ds(base + i * ch, ch)],
                             erows.at[b], esem)
            pltpu.async_copy(dr_hbm.at[didx.at[i]], drrows.at[b], dsem)

        for b in range(nb):
            fire(b, b)

        def outer(g, carry):
            for b in range(nb):
                i = g * nb + b
                pltpu.make_async_copy(hn_hbm.at[pl.ds(0, ch)],
                                      hrows.at[b], hsem).wait()
                pltpu.make_async_copy(ex_hbm.at[pl.ds(0, ch)],
                                      erows.at[b], esem).wait()
                pltpu.make_async_copy(dr_hbm.at[pl.ds(0, ch)],
                                      drrows.at[b], dsem).wait()

                @pl.when(g > 0)
                def _():
                    pltpu.make_async_copy(hn_hbm.at[pl.ds(0, ch)],
                                          mrows.at[b], ssem).wait()

                def edge(e, carry2):
                    av = erows[b, e] * drrows[b, e]
                    for hh in range(heads):
                        m = _lane_bcast(av, hh)
                        for v in range(vregs_per_head):
                            off = (hh * vregs_per_head + v) * 16
                            mrows[b, e, pl.ds(off, 16)] = (
                                hrows[b, e, pl.ds(off, 16)] * m)
                    return carry2

                lax.fori_loop(0, ch, edge, 0, unroll=4)
                nxt = jnp.minimum(i + nb, nchunks - 1)
                fire(nxt, b)
                pltpu.async_copy(mrows.at[b], acc.at[didx.at[i]], ssem,
                                 add=True)
            return carry

        lax.fori_loop(0, nchunks // nb, outer, 0)
        for b in range(nb):
            pltpu.make_async_copy(hn_hbm.at[pl.ds(0, ch)],
                                  hrows.at[b], hsem).wait()
            pltpu.make_async_copy(ex_hbm.at[pl.ds(0, ch)],
                                  erows.at[b], esem).wait()
            pltpu.make_async_copy(dr_hbm.at[pl.ds(0, ch)],
                                  drrows.at[b], dsem).wait()
            pltpu.make_async_copy(hn_hbm.at[pl.ds(0, ch)],
                                  mrows.at[b], ssem).wait()
        plsc.subcore_barrier()
        pltpu.sync_copy(acc.at[pl.ds(s * rpt, rpt)],
                        out_hbm.at[c].at[pl.ds(s * rpt, rpt)])

    return msg_k


def _sc_msg(hn, ex16, dr16, src48, dst48, n_out, heads):
    m_pad = src48.shape[0] * 48
    return _make_msg(n_out, m_pad, heads)(hn, ex16, dr16, src48, dst48)


def _sc_scatter_add(vals, idx2d, n_out):
    m, r = vals.shape
    assert m % GRAN == 0
    ch = 64 if r * 4 > 256 else CH
    assert idx2d.shape == (m // ch, ch)
    return _make_scatter_add(n_out, r, m)(vals, idx2d)


def _full(shape):
    return pl.BlockSpec(shape, lambda i: tuple(0 for _ in shape))


def _rows(bshape):
    return pl.BlockSpec(bshape, lambda i: (i,) + tuple(0 for _ in bshape[1:]))


def _tc_mask_x(xg, tflag, mask_token):
    n = tflag.shape[0]
    b = 1000

    def body(xg_ref, tf_ref, mt_ref, out_ref):
        tf = tf_ref[...]
        out_ref[...] = xg_ref[...] * (1.0 - tf) + tf * mt_ref[...]

    return pl.pallas_call(
        body,
        grid=(n // b,),
        in_specs=[_rows((b, D)), _rows((b, 1)), _full((1, D))],
        out_specs=_rows((b, D)),
        out_shape=jax.ShapeDtypeStruct((n, D), _f32),
    )(xg, tflag, mask_token)


def _tc_pre(h, wn, wni, wnj):
    n, d = h.shape
    b = 1000
    dh = wn.shape[1]
    de16 = wni.shape[1]

    def body(h_ref, wn_ref, wni_ref, wnj_ref, hn_ref, fi_ref, fj_ref):
        hb = h_ref[...]
        hn_ref[...] = jnp.dot(hb, wn_ref[...], preferred_element_type=_f32)
        fi_ref[...] = jnp.dot(hb, wni_ref[...], preferred_element_type=_f32)
        fj_ref[...] = jnp.dot(hb, wnj_ref[...], preferred_element_type=_f32)

    return pl.pallas_call(
        body,
        grid=(n // b,),
        in_specs=[_rows((b, d)), _full((d, dh)), _full((d, de16)), _full((d, de16))],
        out_specs=[_rows((b, dh)), _rows((b, de16)), _rows((b, de16))],
        out_shape=[
            jax.ShapeDtypeStruct((n, dh), _f32),
            jax.ShapeDtypeStruct((n, de16), _f32),
            jax.ShapeDtypeStruct((n, de16), _f32),
        ],
    )(h, wn, wni, wnj)


def _edge_iota(b, pid):
    row = lax.broadcasted_iota(_i32, (b, 128), 0) + pid * b
    lane = lax.broadcasted_iota(_i32, (b, 128), 1)
    return row * 8 + lane // 16, lane % 16


def _tc_matmul(a, w, zero_from=None):
    m = a.shape[0]
    ap = a.reshape(m // 8, 128)
    w8 = jnp.kron(jnp.eye(8, dtype=_f32), w)
    b = 1024
    assert (m // 8) % b == 0

    def body(a_ref, w_ref, o_ref):
        o = jnp.dot(a_ref[...], w_ref[...], preferred_element_type=_f32)
        if zero_from is not None:
            edge, _ = _edge_iota(b, pl.program_id(0))
            o = jnp.where(edge < zero_from, o, 0.0)
        o_ref[...] = o

    out = pl.pallas_call(
        body,
        grid=(m // 8 // b,),
        in_specs=[_rows((b, 128)), _full((128, 128))],
        out_specs=_rows((b, 128)),
        out_shape=jax.ShapeDtypeStruct((m // 8, 128), _f32),
    )(ap, w8)
    return out.reshape(m, 16)


def _tc_edge(fi_g, fj_g, fe, s_mat, heads, e_real, ln_s=None, ln_b=None):
    m = fi_g.shape[0]
    b = 1024
    with_en = ln_s is not None
    mp = m // 8
    assert mp % b == 0

    def body(*refs):
        if with_en:
            fi_ref, fj_ref, fe_ref, s_ref, mt_ref, lns_ref, lnb_ref, \
                ex_ref, en_ref = refs
        else:
            fi_ref, fj_ref, fe_ref, s_ref, ex_ref = refs
        f = fi_ref[...] + fj_ref[...] + fe_ref[...]
        f = jnp.where(f > 0, f, 0.2 * f)
        sc = jnp.dot(f, s_ref[...], preferred_element_type=_f32)
        edge, col = _edge_iota(b, pl.program_id(0))
        keep = jnp.logical_and(edge < e_real, col < heads)
        ex_ref[...] = jnp.where(keep, jnp.exp(sc), 0.0)
        if with_en:
            r0 = jnp.maximum(f, 0.0)
            mt = mt_ref[...]
            mu = jnp.dot(r0, mt, preferred_element_type=_f32)
            var = jnp.dot(r0 * r0, mt, preferred_element_type=_f32) - mu * mu
            en_ref[...] = ((r0 - mu) * lax.rsqrt(var + 1e-5) * lns_ref[...]
                           + lnb_ref[...])

    s8 = jnp.kron(jnp.eye(8, dtype=_f32), s_mat)
    in_specs = [_rows((b, 128))] * 3 + [_full((128, 128))]
    out_specs = [_rows((b, 128))]
    out_shape = [jax.ShapeDtypeStruct((mp, 128), _f32)]
    args = [fi_g.reshape(mp, 128), fj_g.reshape(mp, 128),
            fe.reshape(mp, 128), s8]
    if with_en:
        mt8 = jnp.asarray(np.kron(np.eye(8, dtype=np.float32),
                                  np.full((16, 16), 1.0 / 16, np.float32)))
        in_specs += [_full((128, 128)), _full((1, 128)), _full((1, 128))]
        args += [mt8, jnp.tile(ln_s, 8).reshape(1, 128),
                 jnp.tile(ln_b, 8).reshape(1, 128)]
        out_specs.append(_rows((b, 128)))
        out_shape.append(jax.ShapeDtypeStruct((mp, 128), _f32))
    out = pl.pallas_call(
        body,
        grid=(mp // b,),
        in_specs=in_specs,
        out_specs=out_specs if with_en else out_specs[0],
        out_shape=out_shape if with_en else out_shape[0],
    )(*args)
    if with_en:
        return out[0].reshape(m, 16), out[1].reshape(m, 16)
    return out.reshape(m, 16), None


def _tc_recip_sum(dparts):
    n = dparts.shape[1]

    def body(d0_ref, d1_ref, o_ref):
        o_ref[...] = 1.0 / (d0_ref[...] + d1_ref[...] + 1e-9)

    np8 = n // 8
    out = pl.pallas_call(
        body,
        grid=(1,),
        in_specs=[_rows((np8, 128)), _rows((np8, 128))],
        out_specs=_rows((np8, 128)),
        out_shape=jax.ShapeDtypeStruct((np8, 128), _f32),
    )(dparts[0].reshape(np8, 128), dparts[1].reshape(np8, 128))
    return out.reshape(n, 16)


def _tc_msg(ex16, dgr, hn_g, x_mat):
    m = hn_g.shape[0]
    b = 2048

    def body(ex_ref, dg_ref, hn_ref, x_ref, o_ref):
        a = jnp.dot(ex_ref[...] * dg_ref[...], x_ref[...],
                    preferred_element_type=_f32)
        o_ref[...] = a * hn_ref[...]

    return pl.pallas_call(
        body,
        grid=(m // b,),
        in_specs=[_rows((b, 16)), _rows((b, 16)), _rows((b, D)), _full((16, D))],
        out_specs=_rows((b, D)),
        out_shape=jax.ShapeDtypeStruct((m, D), _f32),
    )(ex16, dgr, hn_g, x_mat)


def _tc_post(mparts, h, ln_s, ln_b):
    n = h.shape[0]
    b = 1000

    def body(p0_ref, p1_ref, h_ref, s_ref, bb_ref, o_ref):
        o = p0_ref[...] + p1_ref[...] + h_ref[...]
        o = jnp.maximum(o, 0.0)
        mu = jnp.mean(o, axis=-1, keepdims=True)
        var = jnp.mean(o * o, axis=-1, keepdims=True) - mu * mu
        o_ref[...] = (o - mu) * lax.rsqrt(var + 1e-5) * s_ref[...] + bb_ref[...]

    return pl.pallas_call(
        body,
        grid=(n // b,),
        in_specs=[_rows((b, D))] * 3 + [_full((1, D)), _full((1, D))],
        out_specs=_rows((b, D)),
        out_shape=jax.ShapeDtypeStruct((n, D), _f32),
    )(mparts[0], mparts[1], h, ln_s.reshape(1, D), ln_b.reshape(1, D))


def _tc_rep(h, w, mflag):
    n = h.shape[0]
    b = 1000

    def body(h_ref, w_ref, mf_ref, o_ref):
        o = jnp.dot(h_ref[...], w_ref[...], preferred_element_type=_f32)
        o_ref[...] = o * (1.0 - mf_ref[...])

    return pl.pallas_call(
        body,
        grid=(n // b,),
        in_specs=[_rows((b, D)), _full((D, D)), _rows((b, 1))],
        out_specs=_rows((b, D)),
        out_shape=jax.ShapeDtypeStruct((n, D), _f32),
    )(h, w, mflag)


def _tc_loss(mparts, rep, x, mflag, num_mask):
    n = x.shape[0]
    b = 1000

    def body(p0_ref, p1_ref, rep_ref, x_ref, mf_ref, o_ref):
        recon = p0_ref[...] + p1_ref[...] + rep_ref[...]
        xb = x_ref[...]
        nx = jnp.sqrt(jnp.sum(recon * recon, -1, keepdims=True)) + 1e-8
        ny = jnp.sqrt(jnp.sum(xb * xb, -1, keepdims=True)) + 1e-8
        cos = jnp.sum((recon / nx) * (xb / ny), -1, keepdims=True)
        v = (1.0 - cos) ** 3 * mf_ref[...]
        psum = jnp.sum(v, axis=0, keepdims=True)

        @pl.when(pl.program_id(0) == 0)
        def _():
            o_ref[...] = jnp.zeros((1, 1), _f32)

        o_ref[...] += psum

    out = pl.pallas_call(
        body,
        grid=(n // b,),
        in_specs=[_rows((b, D))] * 4 + [_rows((b, 1))],
        out_specs=pl.BlockSpec((1, 1), lambda i: (0, 0)),
        out_shape=jax.ShapeDtypeStruct((1, 1), _f32),
    )(mparts[0], mparts[1], rep, x, mflag)
    return out[0, 0] / np.float32(num_mask)


def _build_s(attn, de):
    attn_flat = attn.reshape(-1)
    rows = np.arange(16)
    s = jnp.zeros((16, 16), _f32).at[rows, rows // de].set(attn_flat)
    return s


@functools.lru_cache(maxsize=None)
def _build_x_mat(heads, dn):
    x = np.zeros((16, heads * dn), np.float32)
    for hh in range(heads):
        x[hh, hh * dn:(hh + 1) * dn] = 1.0
    return x


def _egat(h, ef, idxs, lp, heads, de, e_real, enc=True):
    src128, dst128, src32, dst32 = idxs
    hn, fi, fj = _tc_pre(h, lp["Wn"], lp["Wni"], lp["Wnj"])
    fe = _tc_matmul(ef, lp["We"], zero_from=None)
    fi_g = _sc_gather(fi, src128)
    fj_g = _sc_gather(fj, dst128)
    s_mat = _build_s(lp["attn"], de)
    if enc:
        ex16, en = _tc_edge(fi_g, fj_g, fe, s_mat, heads, e_real,
                            lp["ln_e_s"], lp["ln_e_b"])
    else:
        ex16, en = _tc_edge(fi_g, fj_g, fe, s_mat, heads, e_real)
    dparts = _sc_scatter_add(ex16, dst128, N)
    dr16 = _tc_recip_sum(dparts)
    mparts = _sc_msg(hn, ex16, dr16, src32, dst32, N, heads)
    return mparts, en


def kernel(x, edge_index, e, params):
    n = x.shape[0]
    plan = _static_plan(n, edge_index.shape[1])
    k, e_real, ep = plan["k"], plan["e_real"], plan["ep"]

    tflag = jnp.asarray(plan["tflag"])
    mflag = jnp.asarray(plan["mflag"])

    pairs = jnp.concatenate(
        [edge_index.T, jnp.asarray(plan["loop_pairs"])], axis=0)
    table16 = jnp.pad(pairs, ((0, 0), (0, 14)))
    pairs_p = _sc_gather(table16, jnp.asarray(plan["pairs_idx2d"]))
    src_p = pairs_p[:, 0]
    dst_p = pairs_p[:, 1]
    src128 = src_p.reshape(-1, CH)
    dst128 = dst_p.reshape(-1, CH)
    src32 = src_p.reshape(-1, 48)
    dst32 = dst_p.reshape(-1, 48)
    idxs = (src128, dst128, src32, dst32)

    xg = _sc_gather(x, jnp.asarray(plan["rowsrc2d"]))
    h = _tc_mask_x(xg, tflag, params["mask_token"])

    e_g = _sc_gather(e, jnp.asarray(plan["eidx2d"]))

    lp = params["enc0"]
    hn0, fi0, fj0 = _tc_pre(h, lp["Wn"], lp["Wni"], lp["Wnj"])
    fe0 = _tc_matmul(e_g, lp["We"], zero_from=k)
    fi0_g = _sc_gather(fi0, src128)
    fj0_g = _sc_gather(fj0, dst128)
    ex0, en0 = _tc_edge(fi0_g, fj0_g, fe0, _build_s(lp["attn"], DE_H), H, e_real,
                        lp["ln_e_s"], lp["ln_e_b"])
    d0 = _sc_scatter_add(ex0, dst128, N)
    dr0 = _tc_recip_sum(d0)
    mp0 = _sc_msg(hn0, ex0, dr0, src32, dst32, N, H)
    h = _tc_post(mp0, h, lp["ln_n_s"], lp["ln_n_b"])
    ef = en0

    lp = params["enc1"]
    mp1, en1 = _egat(h, ef, idxs, lp, H, DE_H, e_real, enc=True)
    h = _tc_post(mp1, h, lp["ln_n_s"], lp["ln_n_b"])
    ef = en1

    rep = _tc_rep(h, params["W_e2d"], mflag)
    rep_e = _tc_matmul(ef, params["W_e2d_e"])
    mpd, _ = _egat(rep, rep_e, idxs, params["dec"], 1, DE, e_real, enc=False)
    return _tc_loss(mpd, rep, x, mflag, plan["num_mask"])
